# Initial kernel scaffold; baseline (speedup 1.0000x reference)
import jax
import jax.numpy as jnp
from jax.experimental import pallas as pl


def kernel(x, edge_index, batch, params):
    raise NotImplementedError("write your pallas kernel here")


# speedup vs baseline: 1.0364x; 1.0364x over previous
import functools
import math

import jax
import jax.numpy as jnp
from jax import lax
from jax.experimental import pallas as pl
from jax.experimental.pallas import tpu as pltpu

N_NODES = 10000
N_EDGES = 320000
IN_CH = 128
HID = 128
OUT_CH = 64
NUM_HEADS = 4
SEEDS1 = 75

_BN_SCALE = 1.0 / math.sqrt(1.0 + 1e-5)


def _linear_body(x_ref, w_ref, b_ref, o_ref):
    o_ref[...] = lax.dot_general(
        x_ref[...], w_ref[...], (((1,), (1,)), ((), ())),
        preferred_element_type=jnp.float32) + b_ref[...]


def _linear_tc(x, w, b, blk=1000):
    n, _ = x.shape
    od, idim = w.shape
    return pl.pallas_call(
        _linear_body,
        grid=(n // blk,),
        in_specs=[
            pl.BlockSpec((blk, idim), lambda i: (i, 0)),
            pl.BlockSpec((od, idim), lambda i: (0, 0)),
            pl.BlockSpec((1, od), lambda i: (0, 0)),
        ],
        out_specs=pl.BlockSpec((blk, od), lambda i: (i, 0)),
        out_shape=jax.ShapeDtypeStruct((n, od), jnp.float32),
    )(x, w, b[None])


def _bn_eval(x, g, b):
    return x * (_BN_SCALE * g) + b


def _ln(x, g, b):
    m = jnp.mean(x, axis=-1, keepdims=True)
    v = jnp.mean((x - m) ** 2, axis=-1, keepdims=True)
    return (x - m) / jnp.sqrt(v + 1e-5) * g + b


def _mlp(x, p, pre):
    x = x @ p[pre + '_l0_w'].T + p[pre + '_l0_b']
    x = _bn_eval(x, p[pre + '_bn0_g'], p[pre + '_bn0_b'])
    x = jax.nn.relu(x)
    x = x @ p[pre + '_l1_w'].T + p[pre + '_l1_b']
    x = _bn_eval(x, p[pre + '_bn1_g'], p[pre + '_bn1_b'])
    return x


def _gin_conv(x, src, dst, p, pre):
    agg = jax.ops.segment_sum(x[src], dst, num_segments=x.shape[0])
    return _mlp(x + agg, p, pre)


def _gat_conv(x, src, dst, w, att_src, att_dst, bias):
    n = x.shape[0]
    loop = jnp.arange(n, dtype=src.dtype)
    s = jnp.concatenate([src, loop])
    d = jnp.concatenate([dst, loop])
    h = x @ w.T
    a_src = jnp.sum(h * att_src, axis=-1)
    a_dst = jnp.sum(h * att_dst, axis=-1)
    alpha = jax.nn.leaky_relu(a_src[s] + a_dst[d], 0.2)
    amax = jax.ops.segment_max(lax.stop_gradient(alpha), d, num_segments=n)
    amax = jnp.where(jnp.isfinite(amax), amax, 0.0)
    e = jnp.exp(alpha - amax[d])
    denom = jax.ops.segment_sum(e, d, num_segments=n)
    coef = e / (denom[d] + 1e-16)
    out = jax.ops.segment_sum(coef[:, None] * h[s], d, num_segments=n)
    return out + bias


def _mab(Q_in, K, V, p, pre):
    Q = Q_in @ p[pre + '_fcq_w'].T + p[pre + '_fcq_b']

    def split_heads(t):
        return jnp.concatenate(jnp.split(t, NUM_HEADS, axis=2), axis=0)

    Q_ = split_heads(Q)
    K_ = split_heads(K)
    V_ = split_heads(V)
    score = jnp.einsum('bqd,bkd->bqk', Q_, K_) / math.sqrt(HID)
    A = jax.nn.softmax(score, axis=-1)
    out = Q_ + jnp.einsum('bqk,bkd->bqd', A, V_)
    out = jnp.concatenate(jnp.split(out, NUM_HEADS, axis=0), axis=2)
    out = _ln(out, p[pre + '_ln0_g'], p[pre + '_ln0_b'])
    out = out + jax.nn.relu(out @ p[pre + '_fco_w'].T + p[pre + '_fco_b'])
    out = _ln(out, p[pre + '_ln1_g'], p[pre + '_ln1_b'])
    return out


def kernel(x, edge_index, batch, params):
    p = params
    src = edge_index[0]
    dst = edge_index[1]
    h = _linear_tc(x, p['enc_w'], p['enc_b'])
    h = _gin_conv(h, src, dst, p, 'c1')
    h = jax.nn.relu(h)
    h = _gin_conv(h, src, dst, p, 'c2')
    xg = h @ p['gmt_lin1_w'].T + p['gmt_lin1_b']
    K = _gat_conv(xg, src, dst, p['gatk_lin_w'], p['gatk_att_src'],
                  p['gatk_att_dst'], p['gatk_bias'])[None]
    V = _gat_conv(xg, src, dst, p['gatv_lin_w'], p['gatv_att_src'],
                  p['gatv_att_dst'], p['gatv_bias'])[None]
    S = jnp.broadcast_to(p['pma1_S'], (1, SEEDS1, HID))
    bx = _mab(S, K, V, p, 'mab1')
    K2 = bx @ p['mab2_lk_w'].T + p['mab2_lk_b']
    V2 = bx @ p['mab2_lv_w'].T + p['mab2_lv_b']
    bx = _mab(bx, K2, V2, p, 'mab2')
    K3 = bx @ p['mab3_lk_w'].T + p['mab3_lk_b']
    V3 = bx @ p['mab3_lv_w'].T + p['mab3_lv_b']
    bx = _mab(p['pma2_S'], K3, V3, p, 'mab3')
    out = bx[:, 0, :] @ p['gmt_lin2_w'].T + p['gmt_lin2_b']
    out = out @ p['clf_w'].T + p['clf_b']
    return out


# speedup vs baseline: 1.1306x; 1.0908x over previous
import functools
import math

import jax
import jax.numpy as jnp
from jax import lax
from jax.experimental import pallas as pl
from jax.experimental.pallas import tpu as pltpu
from jax.experimental.pallas import tpu_sc as plsc

N_NODES = 10000
N_EDGES = 320000
IN_CH = 128
HID = 128
OUT_CH = 64
NUM_HEADS = 4
SEEDS1 = 75

_BN_SCALE = 1.0 / math.sqrt(1.0 + 1e-5)


def _linear_body(x_ref, w_ref, b_ref, o_ref):
    o_ref[...] = lax.dot_general(
        x_ref[...], w_ref[...], (((1,), (1,)), ((), ())),
        preferred_element_type=jnp.float32) + b_ref[...]


def _linear_tc(x, w, b, blk=1000):
    n, _ = x.shape
    od, idim = w.shape
    return pl.pallas_call(
        _linear_body,
        grid=(n // blk,),
        in_specs=[
            pl.BlockSpec((blk, idim), lambda i: (i, 0)),
            pl.BlockSpec((od, idim), lambda i: (0, 0)),
            pl.BlockSpec((1, od), lambda i: (0, 0)),
        ],
        out_specs=pl.BlockSpec((blk, od), lambda i: (i, 0)),
        out_shape=jax.ShapeDtypeStruct((n, od), jnp.float32),
    )(x, w, b[None])


_NC = 2
_NS = 16
_NW = _NC * _NS
_CHUNK = 128
_CPW = 80
_N_CHUNKS = _NW * _CPW
_E_PAD = _N_CHUNKS * _CHUNK
_ACC_ROWS = 10240

_SC_MESH = plsc.VectorSubcoreMesh(core_axis_name="c", subcore_axis_name="s")


def _sc_segsum_body(h_hbm, srcp_hbm, dstp_hbm, out_hbm, srcb, dstb, gbuf,
                    acc, sem):
    cid = lax.axis_index("c")
    sid = lax.axis_index("s")
    wid = cid * _NS + sid
    z16 = jnp.zeros((16,), jnp.float32)

    def zrow(i, _):
        for j in range(8):
            gbuf[i, pl.ds(16 * j, 16)] = z16
        return 0

    lax.fori_loop(0, _CHUNK, zrow, 0)

    def zacc(k, _):
        pltpu.sync_copy(gbuf, acc.at[pl.ds(sid * 640 + k * 128, 128)])
        return 0

    lax.fori_loop(0, 5, zacc, 0)
    plsc.subcore_barrier()

    pltpu.sync_copy(srcp_hbm.at[pl.ds(wid * _CPW, _CPW)], srcb)
    pltpu.sync_copy(dstp_hbm.at[pl.ds(wid * _CPW, _CPW)], dstb)

    def step(j, _):
        pltpu.async_copy(h_hbm.at[srcb.at[j]], gbuf, sem).wait()
        pltpu.sync_copy(gbuf, acc.at[dstb.at[j]], add=True)
        return 0

    lax.fori_loop(0, _CPW, step, 0)
    plsc.subcore_barrier()
    pltpu.sync_copy(acc.at[pl.ds(sid * 640, 640)],
                    out_hbm.at[cid, pl.ds(sid * 640, 640)])


def _sc_segsum(h, srcp, dstp):
    f = pl.kernel(
        _sc_segsum_body,
        out_type=jax.ShapeDtypeStruct((_NC, _ACC_ROWS, HID), jnp.float32),
        mesh=_SC_MESH,
        scratch_types=[
            pltpu.VMEM((_CPW, _CHUNK), jnp.int32),
            pltpu.VMEM((_CPW, _CHUNK), jnp.int32),
            pltpu.VMEM((_CHUNK, HID), jnp.float32),
            pltpu.MemorySpace.VMEM_SHARED((_ACC_ROWS, HID), jnp.float32),
            pltpu.SemaphoreType.DMA,
        ],
    )
    return f(h, srcp, dstp)


def _pad_edges(src, dst):
    npad = _E_PAD - N_EDGES
    srcp = jnp.concatenate(
        [src, jnp.zeros((npad,), jnp.int32)]).reshape(_N_CHUNKS, _CHUNK)
    dstp = jnp.concatenate(
        [dst, jnp.full((npad,), N_NODES, jnp.int32)]).reshape(
            _N_CHUNKS, _CHUNK)
    return srcp, dstp


def _bn_eval(x, g, b):
    return x * (_BN_SCALE * g) + b


def _ln(x, g, b):
    m = jnp.mean(x, axis=-1, keepdims=True)
    v = jnp.mean((x - m) ** 2, axis=-1, keepdims=True)
    return (x - m) / jnp.sqrt(v + 1e-5) * g + b


def _mlp(x, p, pre):
    x = x @ p[pre + '_l0_w'].T + p[pre + '_l0_b']
    x = _bn_eval(x, p[pre + '_bn0_g'], p[pre + '_bn0_b'])
    x = jax.nn.relu(x)
    x = x @ p[pre + '_l1_w'].T + p[pre + '_l1_b']
    x = _bn_eval(x, p[pre + '_bn1_g'], p[pre + '_bn1_b'])
    return x


def _gin_conv(x, srcp, dstp, p, pre):
    parts = _sc_segsum(x, srcp, dstp)
    return _mlp(x + parts[0, :N_NODES] + parts[1, :N_NODES], p, pre)


def _gat_conv(x, src, dst, w, att_src, att_dst, bias):
    n = x.shape[0]
    loop = jnp.arange(n, dtype=src.dtype)
    s = jnp.concatenate([src, loop])
    d = jnp.concatenate([dst, loop])
    h = x @ w.T
    a_src = jnp.sum(h * att_src, axis=-1)
    a_dst = jnp.sum(h * att_dst, axis=-1)
    alpha = jax.nn.leaky_relu(a_src[s] + a_dst[d], 0.2)
    amax = jax.ops.segment_max(lax.stop_gradient(alpha), d, num_segments=n)
    amax = jnp.where(jnp.isfinite(amax), amax, 0.0)
    e = jnp.exp(alpha - amax[d])
    denom = jax.ops.segment_sum(e, d, num_segments=n)
    coef = e / (denom[d] + 1e-16)
    out = jax.ops.segment_sum(coef[:, None] * h[s], d, num_segments=n)
    return out + bias


def _mab(Q_in, K, V, p, pre):
    Q = Q_in @ p[pre + '_fcq_w'].T + p[pre + '_fcq_b']

    def split_heads(t):
        return jnp.concatenate(jnp.split(t, NUM_HEADS, axis=2), axis=0)

    Q_ = split_heads(Q)
    K_ = split_heads(K)
    V_ = split_heads(V)
    score = jnp.einsum('bqd,bkd->bqk', Q_, K_) / math.sqrt(HID)
    A = jax.nn.softmax(score, axis=-1)
    out = Q_ + jnp.einsum('bqk,bkd->bqd', A, V_)
    out = jnp.concatenate(jnp.split(out, NUM_HEADS, axis=0), axis=2)
    out = _ln(out, p[pre + '_ln0_g'], p[pre + '_ln0_b'])
    out = out + jax.nn.relu(out @ p[pre + '_fco_w'].T + p[pre + '_fco_b'])
    out = _ln(out, p[pre + '_ln1_g'], p[pre + '_ln1_b'])
    return out


def kernel(x, edge_index, batch, params):
    p = params
    src = edge_index[0]
    dst = edge_index[1]
    srcp, dstp = _pad_edges(src, dst)
    h = _linear_tc(x, p['enc_w'], p['enc_b'])
    h = _gin_conv(h, srcp, dstp, p, 'c1')
    h = jax.nn.relu(h)
    h = _gin_conv(h, srcp, dstp, p, 'c2')
    xg = h @ p['gmt_lin1_w'].T + p['gmt_lin1_b']
    K = _gat_conv(xg, src, dst, p['gatk_lin_w'], p['gatk_att_src'],
                  p['gatk_att_dst'], p['gatk_bias'])[None]
    V = _gat_conv(xg, src, dst, p['gatv_lin_w'], p['gatv_att_src'],
                  p['gatv_att_dst'], p['gatv_bias'])[None]
    S = jnp.broadcast_to(p['pma1_S'], (1, SEEDS1, HID))
    bx = _mab(S, K, V, p, 'mab1')
    K2 = bx @ p['mab2_lk_w'].T + p['mab2_lk_b']
    V2 = bx @ p['mab2_lv_w'].T + p['mab2_lv_b']
    bx = _mab(bx, K2, V2, p, 'mab2')
    K3 = bx @ p['mab3_lk_w'].T + p['mab3_lk_b']
    V3 = bx @ p['mab3_lv_w'].T + p['mab3_lv_b']
    bx = _mab(p['pma2_S'], K3, V3, p, 'mab3')
    out = bx[:, 0, :] @ p['gmt_lin2_w'].T + p['gmt_lin2_b']
    out = out @ p['clf_w'].T + p['clf_b']
    return out


# speedup vs baseline: 6.2010x; 5.4849x over previous
import functools
import math

import jax
import jax.numpy as jnp
from jax import lax
from jax.experimental import pallas as pl
from jax.experimental.pallas import tpu as pltpu
from jax.experimental.pallas import tpu_sc as plsc

N_NODES = 10000
N_EDGES = 320000
IN_CH = 128
HID = 128
OUT_CH = 64
NUM_HEADS = 4
SEEDS1 = 75

_BN_SCALE = 1.0 / math.sqrt(1.0 + 1e-5)


def _linear_body(x_ref, w_ref, b_ref, o_ref):
    o_ref[...] = lax.dot_general(
        x_ref[...], w_ref[...], (((1,), (1,)), ((), ())),
        preferred_element_type=jnp.float32) + b_ref[...]


def _linear_tc(x, w, b, blk=1000):
    n, _ = x.shape
    od, idim = w.shape
    return pl.pallas_call(
        _linear_body,
        grid=(n // blk,),
        in_specs=[
            pl.BlockSpec((blk, idim), lambda i: (i, 0)),
            pl.BlockSpec((od, idim), lambda i: (0, 0)),
            pl.BlockSpec((1, od), lambda i: (0, 0)),
        ],
        out_specs=pl.BlockSpec((blk, od), lambda i: (i, 0)),
        out_shape=jax.ShapeDtypeStruct((n, od), jnp.float32),
    )(x, w, b[None])


_NC = 2
_NS = 16
_NW = _NC * _NS
_CHUNK = 128
_CPW = 80
_N_CHUNKS = _NW * _CPW
_E_PAD = _N_CHUNKS * _CHUNK
_ACC_ROWS = 10240

_SC_MESH = plsc.VectorSubcoreMesh(core_axis_name="c", subcore_axis_name="s")


def _edge_weights(srcb, dstb, asb, adb, amax, j, t):
    s16 = srcb.at[j][pl.ds(16 * t, 16)]
    d16 = dstb.at[j][pl.ds(16 * t, 16)]
    a_s = plsc.load_gather(asb, [s16])
    a_d = plsc.load_gather(adb, [d16])
    al = a_s + a_d
    al = jnp.maximum(al, 0.2 * al)
    c = a_d + amax
    c = jnp.maximum(c, 0.2 * c)
    return jnp.exp(al - c)


def _sc_edge_body(h_hbm, asrc_hbm, adst_hbm, amax_hbm, srcp_hbm, dstp_hbm,
                  out_hbm, srcb, dstb, gbuf, asb, adb, amaxb, wbuf, acc, sem):
    cid = lax.axis_index("c")
    sid = lax.axis_index("s")
    wid = cid * _NS + sid
    z16 = jnp.zeros((16,), jnp.float32)

    def zrow(i, _):
        for j in range(8):
            gbuf.at[i][pl.ds(16 * j, 16)] = z16
        return 0

    lax.fori_loop(0, _CHUNK, zrow, 0)

    def zacc(k, _):
        pltpu.sync_copy(gbuf, acc.at[pl.ds(sid * 640 + k * 128, 128)])
        return 0

    lax.fori_loop(0, 5, zacc, 0)

    pltpu.sync_copy(asrc_hbm, asb)
    pltpu.sync_copy(adst_hbm, adb)
    pltpu.sync_copy(amax_hbm, amaxb)
    plsc.subcore_barrier()

    amax = amaxb[...]

    def slab(g, _):
        pltpu.sync_copy(srcp_hbm.at[pl.ds(wid * _CPW + g * 8, 8)], srcb)
        pltpu.sync_copy(dstp_hbm.at[pl.ds(wid * _CPW + g * 8, 8)], dstb)

        def step(j, _):
            cp = pltpu.async_copy(h_hbm.at[srcb.at[j]], gbuf, sem)
            for t in range(8):
                w = _edge_weights(srcb, dstb, asb, adb, amax, j, t)
                wbuf[pl.ds(16 * t, 16)] = w
            cp.wait()

            def scale(e, _):
                ev = jnp.broadcast_to(e, (16,)).astype(jnp.int32)
                we = plsc.load_gather(wbuf, [ev])
                for t in range(8):
                    gbuf.at[e][pl.ds(16 * t, 16)] = (
                        gbuf.at[e][pl.ds(16 * t, 16)] * we)
                return 0

            lax.fori_loop(0, _CHUNK, scale, 0)
            pltpu.sync_copy(gbuf, acc.at[dstb.at[j]], add=True)
            return 0

        lax.fori_loop(0, 8, step, 0)
        return 0

    lax.fori_loop(0, _CPW // 8, slab, 0)
    plsc.subcore_barrier()
    pltpu.sync_copy(acc.at[pl.ds(sid * 640, 640)],
                    out_hbm.at[cid, pl.ds(sid * 640, 640)])


def _sc_edge_pass(h, a_src, a_dst, amax, srcp, dstp):
    f = pl.kernel(
        _sc_edge_body,
        out_type=jax.ShapeDtypeStruct((_NC, _ACC_ROWS, HID), jnp.float32),
        mesh=_SC_MESH,
        scratch_types=[
            pltpu.VMEM((8, _CHUNK), jnp.int32),
            pltpu.VMEM((8, _CHUNK), jnp.int32),
            pltpu.VMEM((_CHUNK, HID), jnp.float32),
            pltpu.VMEM((N_NODES,), jnp.float32),
            pltpu.VMEM((N_NODES,), jnp.float32),
            pltpu.VMEM((16,), jnp.float32),
            pltpu.VMEM((_CHUNK,), jnp.float32),
            pltpu.MemorySpace.VMEM_SHARED((_ACC_ROWS, HID), jnp.float32),
            pltpu.SemaphoreType.DMA,
        ],
        compiler_params=pltpu.CompilerParams(needs_layout_passes=False),
    )
    return f(h, a_src, a_dst, amax, srcp, dstp)


def _pad_edges(src, dst):
    npad = _E_PAD - N_EDGES
    srcp = jnp.concatenate(
        [src, jnp.zeros((npad,), jnp.int32)]).reshape(_N_CHUNKS, _CHUNK)
    dstp = jnp.concatenate(
        [dst, jnp.full((npad,), N_NODES, jnp.int32)]).reshape(
            _N_CHUNKS, _CHUNK)
    return srcp, dstp


def _bn_eval(x, g, b):
    return x * (_BN_SCALE * g) + b


def _ln(x, g, b):
    m = jnp.mean(x, axis=-1, keepdims=True)
    v = jnp.mean((x - m) ** 2, axis=-1, keepdims=True)
    return (x - m) / jnp.sqrt(v + 1e-5) * g + b


def _mlp(x, p, pre):
    x = x @ p[pre + '_l0_w'].T + p[pre + '_l0_b']
    x = _bn_eval(x, p[pre + '_bn0_g'], p[pre + '_bn0_b'])
    x = jax.nn.relu(x)
    x = x @ p[pre + '_l1_w'].T + p[pre + '_l1_b']
    x = _bn_eval(x, p[pre + '_bn1_g'], p[pre + '_bn1_b'])
    return x


_Z_NODE = None


def _gin_conv(x, srcp, dstp, zn, z16, p, pre):
    parts = _sc_edge_pass(x, zn, zn, z16, srcp, dstp)
    return _mlp(x + parts[0, :N_NODES] + parts[1, :N_NODES], p, pre)


def _after(dep, *xs):
    return lax.optimization_barrier((dep, xs))[1]


def _gat_pre(x, w, att_src, att_dst):
    h = x @ w.T
    a_src = jnp.sum(h * att_src, axis=-1)
    a_dst = jnp.sum(h * att_dst, axis=-1)
    amax = jnp.max(a_src)
    return h, a_src, a_dst, amax


def _gat_post(numparts, den, h, a_src, a_dst, amax, bias):
    num = numparts[0, :N_NODES] + numparts[1, :N_NODES]
    al = jax.nn.leaky_relu(a_src + a_dst, 0.2)
    c = jax.nn.leaky_relu(a_dst + amax, 0.2)
    wl = jnp.exp(al - c)
    ntot = num + wl[:, None] * h
    dtot = den + wl
    return ntot / (dtot + 1e-16)[:, None] + bias


def _mab(Q_in, K, V, p, pre):
    Q = Q_in @ p[pre + '_fcq_w'].T + p[pre + '_fcq_b']

    def split_heads(t):
        return jnp.concatenate(jnp.split(t, NUM_HEADS, axis=2), axis=0)

    Q_ = split_heads(Q)
    K_ = split_heads(K)
    V_ = split_heads(V)
    score = jnp.einsum('bqd,bkd->bqk', Q_, K_) / math.sqrt(HID)
    A = jax.nn.softmax(score, axis=-1)
    out = Q_ + jnp.einsum('bqk,bkd->bqd', A, V_)
    out = jnp.concatenate(jnp.split(out, NUM_HEADS, axis=0), axis=2)
    out = _ln(out, p[pre + '_ln0_g'], p[pre + '_ln0_b'])
    out = out + jax.nn.relu(out @ p[pre + '_fco_w'].T + p[pre + '_fco_b'])
    out = _ln(out, p[pre + '_ln1_g'], p[pre + '_ln1_b'])
    return out


def kernel(x, edge_index, batch, params):
    p = params
    src = edge_index[0]
    dst = edge_index[1]
    srcp, dstp = _pad_edges(src, dst)
    zn = jnp.zeros((N_NODES,), jnp.float32)
    z16 = jnp.zeros((16,), jnp.float32)
    h = _linear_tc(x, p['enc_w'], p['enc_b'])
    h = _gin_conv(h, srcp, dstp, zn, z16, p, 'c1')
    h = jax.nn.relu(h)
    h = _gin_conv(h, srcp, dstp, zn, z16, p, 'c2')
    xg = h @ p['gmt_lin1_w'].T + p['gmt_lin1_b']
    hK, aKs, aKd, amaxK = _gat_pre(xg, p['gatk_lin_w'], p['gatk_att_src'],
                                   p['gatk_att_dst'])
    hV, aVs, aVd, amaxV = _gat_pre(xg, p['gatv_lin_w'], p['gatv_att_src'],
                                   p['gatv_att_dst'])
    ones_h = jnp.ones((N_NODES, HID), jnp.float32)
    amaxK16 = jnp.broadcast_to(amaxK, (16,))
    amaxV16 = jnp.broadcast_to(amaxV, (16,))
    numK = _sc_edge_pass(hK, aKs, aKd, amaxK16, srcp, dstp)
    hV_, aVs_, aVd_, amaxV16_ = _after(numK, hV, aVs, aVd, amaxV16)
    numV = _sc_edge_pass(hV_, aVs_, aVd_, amaxV16_, srcp, dstp)
    onesK, aKs_, aKd_, amaxK16_ = _after(numV, ones_h, aKs, aKd, amaxK16)
    denKp = _sc_edge_pass(onesK, aKs_, aKd_, amaxK16_, srcp, dstp)
    onesV, aVs2, aVd2, amaxV2 = _after(denKp, ones_h, aVs_, aVd_, amaxV16_)
    denVp = _sc_edge_pass(onesV, aVs2, aVd2, amaxV2, srcp, dstp)
    denK = denKp[0, :N_NODES, 0] + denKp[1, :N_NODES, 0]
    denV = denVp[0, :N_NODES, 0] + denVp[1, :N_NODES, 0]
    K = _gat_post(numK, denK, hK, aKs, aKd, amaxK, p['gatk_bias'])[None]
    V = _gat_post(numV, denV, hV, aVs, aVd, amaxV, p['gatv_bias'])[None]
    S = jnp.broadcast_to(p['pma1_S'], (1, SEEDS1, HID))
    bx = _mab(S, K, V, p, 'mab1')
    K2 = bx @ p['mab2_lk_w'].T + p['mab2_lk_b']
    V2 = bx @ p['mab2_lv_w'].T + p['mab2_lv_b']
    bx = _mab(bx, K2, V2, p, 'mab2')
    K3 = bx @ p['mab3_lk_w'].T + p['mab3_lk_b']
    V3 = bx @ p['mab3_lv_w'].T + p['mab3_lv_b']
    bx = _mab(p['pma2_S'], K3, V3, p, 'mab3')
    out = bx[:, 0, :] @ p['gmt_lin2_w'].T + p['gmt_lin2_b']
    out = out @ p['clf_w'].T + p['clf_b']
    return out


# speedup vs baseline: 7.5239x; 1.2133x over previous
import functools
import math

import jax
import jax.numpy as jnp
from jax import lax
from jax.experimental import pallas as pl
from jax.experimental.pallas import tpu as pltpu
from jax.experimental.pallas import tpu_sc as plsc

N_NODES = 10000
N_EDGES = 320000
IN_CH = 128
HID = 128
OUT_CH = 64
NUM_HEADS = 4
SEEDS1 = 75

_BN_SCALE = 1.0 / math.sqrt(1.0 + 1e-5)


def _linear_body(x_ref, w_ref, b_ref, o_ref):
    o_ref[...] = lax.dot_general(
        x_ref[...], w_ref[...], (((1,), (1,)), ((), ())),
        preferred_element_type=jnp.float32) + b_ref[...]


def _linear_tc(x, w, b, blk=1000):
    n, _ = x.shape
    od, idim = w.shape
    return pl.pallas_call(
        _linear_body,
        grid=(n // blk,),
        in_specs=[
            pl.BlockSpec((blk, idim), lambda i: (i, 0)),
            pl.BlockSpec((od, idim), lambda i: (0, 0)),
            pl.BlockSpec((1, od), lambda i: (0, 0)),
        ],
        out_specs=pl.BlockSpec((blk, od), lambda i: (i, 0)),
        out_shape=jax.ShapeDtypeStruct((n, od), jnp.float32),
    )(x, w, b[None])


_NC = 2
_NS = 16
_NW = _NC * _NS
_CHUNK = 64
_SPT = 16
_NSLAB = 10
_CPT = _SPT * _NSLAB
_N_CHUNKS = _NW * _CPT
_E_PAD = _N_CHUNKS * _CHUNK
_ACC_ROWS = 10240

_SC_MESH = plsc.VectorSubcoreMesh(core_axis_name="c", subcore_axis_name="s")


def _edge_weights(sb, db, asb, adb, amax, j, t):
    s16 = sb.at[j][pl.ds(16 * t, 16)]
    d16 = db.at[j][pl.ds(16 * t, 16)]
    a_s = plsc.load_gather(asb, [s16])
    a_d = plsc.load_gather(adb, [d16])
    al = a_s + a_d
    al = jnp.maximum(al, 0.2 * al)
    c = a_d + amax
    c = jnp.maximum(c, 0.2 * c)
    return jnp.exp(al - c)


def _sc_edge_body(h_hbm, asrc_hbm, adst_hbm, amax_hbm, srcp_hbm, dstp_hbm,
                  out_hbm, srcb0, dstb0, srcb1, dstb1, gbuf0, gbuf1,
                  asb, adb, amaxb, wbuf0, wbuf1, dumpb, acc,
                  gsem0, gsem1, ssem0, ssem1, isem0, isem1):
    cid = lax.axis_index("c")
    sid = lax.axis_index("s")
    wid = cid * _NS + sid
    base = wid * _CPT
    z16 = jnp.zeros((16,), jnp.float32)
    dump16 = jnp.full((16,), N_NODES, jnp.int32)

    def zrow(i, _):
        for t in range(8):
            gbuf0.at[i][pl.ds(16 * t, 16)] = z16
            gbuf1.at[i][pl.ds(16 * t, 16)] = z16
        return 0

    lax.fori_loop(0, _CHUNK, zrow, 0)
    for t in range(4):
        dumpb[pl.ds(16 * t, 16)] = dump16

    def zacc(k, _):
        pltpu.sync_copy(gbuf0, acc.at[pl.ds(sid * 640 + k * 64, 64)])
        return 0

    lax.fori_loop(0, 10, zacc, 0)

    pltpu.sync_copy(asrc_hbm, asb)
    pltpu.sync_copy(adst_hbm, adb)
    pltpu.sync_copy(amax_hbm, amaxb)
    pltpu.sync_copy(srcp_hbm.at[pl.ds(base, _SPT)], srcb0)
    pltpu.sync_copy(dstp_hbm.at[pl.ds(base, _SPT)], dstb0)
    plsc.subcore_barrier()
    amax = amaxb[...]

    pltpu.async_copy(gbuf0, acc.at[dumpb], ssem0, add=True)
    pltpu.async_copy(gbuf1, acc.at[dumpb], ssem1, add=True)
    pltpu.async_copy(h_hbm.at[srcb0.at[0]], gbuf0, gsem0)

    bufs = ((gbuf0, wbuf0, gsem0, ssem0), (gbuf1, wbuf1, gsem1, ssem1))

    def chunk_work(sb, db, jj, q, nxt):
        gq, wq, gsem, ssem = bufs[q]
        go, _, gso, _ = bufs[1 - q]
        for t in range(4):
            wq[pl.ds(16 * t, 16)] = _edge_weights(sb, db, asb, adb, amax,
                                                  jj, t)
        pltpu.make_async_copy(h_hbm.at[sb.at[jj]], gq, gsem).wait()
        if nxt is not None:
            nsb, njj = nxt
            pltpu.async_copy(h_hbm.at[nsb.at[njj]], go, gso)
        pltpu.make_async_copy(gq, acc.at[db.at[jj]], ssem).wait()

        def scale(e, _):
            ev = jnp.broadcast_to(e, (16,)).astype(jnp.int32)
            we = plsc.load_gather(wq, [ev])
            for t in range(8):
                gq.at[e][pl.ds(16 * t, 16)] = (
                    gq.at[e][pl.ds(16 * t, 16)] * we)
            return 0

        lax.fori_loop(0, _CHUNK, scale, 0)
        pltpu.async_copy(gq, acc.at[db.at[jj]], ssem, add=True)

    def pair(p, _):
        a0 = base + 2 * p * _SPT
        b0 = a0 + _SPT
        for j in range(_SPT):
            if j == 3:
                pltpu.async_copy(srcp_hbm.at[pl.ds(b0, _SPT)], srcb1, isem1)
                pltpu.async_copy(dstp_hbm.at[pl.ds(b0, _SPT)], dstb1, isem1)
            if j == _SPT - 1:
                pltpu.make_async_copy(srcp_hbm.at[pl.ds(b0, _SPT)], srcb1,
                                      isem1).wait()
                pltpu.make_async_copy(dstp_hbm.at[pl.ds(b0, _SPT)], dstb1,
                                      isem1).wait()
                nxt = (srcb1, 0)
            else:
                nxt = (srcb0, j + 1)
            chunk_work(srcb0, dstb0, j, j % 2, nxt)
        for j in range(_SPT):
            if j == 3:
                @pl.when(p < _NSLAB // 2 - 1)
                def _():
                    nb = base + (2 * p + 2) * _SPT
                    pltpu.async_copy(srcp_hbm.at[pl.ds(nb, _SPT)], srcb0,
                                     isem0)
                    pltpu.async_copy(dstp_hbm.at[pl.ds(nb, _SPT)], dstb0,
                                     isem0)
            if j == _SPT - 1:
                @pl.when(p < _NSLAB // 2 - 1)
                def _():
                    nb = base + (2 * p + 2) * _SPT
                    pltpu.make_async_copy(srcp_hbm.at[pl.ds(nb, _SPT)],
                                          srcb0, isem0).wait()
                    pltpu.make_async_copy(dstp_hbm.at[pl.ds(nb, _SPT)],
                                          dstb0, isem0).wait()
                    chunk_work(srcb1, dstb1, _SPT - 1, (_SPT - 1) % 2,
                               (srcb0, 0))

                @pl.when(p >= _NSLAB // 2 - 1)
                def _():
                    chunk_work(srcb1, dstb1, _SPT - 1, (_SPT - 1) % 2, None)
            else:
                chunk_work(srcb1, dstb1, j, j % 2, (srcb1, j + 1))
        return 0

    lax.fori_loop(0, _NSLAB // 2, pair, 0)

    pltpu.make_async_copy(gbuf0, acc.at[dumpb], ssem0).wait()
    pltpu.make_async_copy(gbuf1, acc.at[dumpb], ssem1).wait()
    plsc.subcore_barrier()
    pltpu.sync_copy(acc.at[pl.ds(sid * 640, 640)],
                    out_hbm.at[cid, pl.ds(sid * 640, 640)])


def _sc_edge_pass(h, a_src, a_dst, amax, srcp, dstp):
    f = pl.kernel(
        _sc_edge_body,
        out_type=jax.ShapeDtypeStruct((_NC, _ACC_ROWS, HID), jnp.float32),
        mesh=_SC_MESH,
        scratch_types=[
            pltpu.VMEM((_SPT, _CHUNK), jnp.int32),
            pltpu.VMEM((_SPT, _CHUNK), jnp.int32),
            pltpu.VMEM((_SPT, _CHUNK), jnp.int32),
            pltpu.VMEM((_SPT, _CHUNK), jnp.int32),
            pltpu.VMEM((_CHUNK, HID), jnp.float32),
            pltpu.VMEM((_CHUNK, HID), jnp.float32),
            pltpu.VMEM((N_NODES,), jnp.float32),
            pltpu.VMEM((N_NODES,), jnp.float32),
            pltpu.VMEM((16,), jnp.float32),
            pltpu.VMEM((_CHUNK,), jnp.float32),
            pltpu.VMEM((_CHUNK,), jnp.float32),
            pltpu.VMEM((_CHUNK,), jnp.int32),
            pltpu.MemorySpace.VMEM_SHARED((_ACC_ROWS, HID), jnp.float32),
            pltpu.SemaphoreType.DMA,
            pltpu.SemaphoreType.DMA,
            pltpu.SemaphoreType.DMA,
            pltpu.SemaphoreType.DMA,
            pltpu.SemaphoreType.DMA,
            pltpu.SemaphoreType.DMA,
        ],
        compiler_params=pltpu.CompilerParams(needs_layout_passes=False),
    )
    return f(h, a_src, a_dst, amax, srcp, dstp)


def _pad_edges(src, dst):
    npad = _E_PAD - N_EDGES
    srcp = jnp.concatenate(
        [src, jnp.zeros((npad,), jnp.int32)]).reshape(_N_CHUNKS, _CHUNK)
    dstp = jnp.concatenate(
        [dst, jnp.full((npad,), N_NODES, jnp.int32)]).reshape(
            _N_CHUNKS, _CHUNK)
    return srcp, dstp


def _bn_eval(x, g, b):
    return x * (_BN_SCALE * g) + b


def _ln(x, g, b):
    m = jnp.mean(x, axis=-1, keepdims=True)
    v = jnp.mean((x - m) ** 2, axis=-1, keepdims=True)
    return (x - m) / jnp.sqrt(v + 1e-5) * g + b


def _mlp(x, p, pre):
    x = x @ p[pre + '_l0_w'].T + p[pre + '_l0_b']
    x = _bn_eval(x, p[pre + '_bn0_g'], p[pre + '_bn0_b'])
    x = jax.nn.relu(x)
    x = x @ p[pre + '_l1_w'].T + p[pre + '_l1_b']
    x = _bn_eval(x, p[pre + '_bn1_g'], p[pre + '_bn1_b'])
    return x


_Z_NODE = None


def _gin_conv(x, srcp, dstp, zn, z16, p, pre):
    parts = _sc_edge_pass(x, zn, zn, z16, srcp, dstp)
    return _mlp(x + parts[0, :N_NODES] + parts[1, :N_NODES], p, pre)


def _after(dep, *xs):
    return lax.optimization_barrier((dep, xs))[1]


def _gat_pre(x, w, att_src, att_dst):
    h = x @ w.T
    a_src = jnp.sum(h * att_src, axis=-1)
    a_dst = jnp.sum(h * att_dst, axis=-1)
    amax = jnp.max(a_src)
    return h, a_src, a_dst, amax


def _gat_post(numparts, den, h, a_src, a_dst, amax, bias):
    num = numparts[0, :N_NODES] + numparts[1, :N_NODES]
    al = jax.nn.leaky_relu(a_src + a_dst, 0.2)
    c = jax.nn.leaky_relu(a_dst + amax, 0.2)
    wl = jnp.exp(al - c)
    ntot = num + wl[:, None] * h
    dtot = den + wl
    return ntot / (dtot + 1e-16)[:, None] + bias


def _mab(Q_in, K, V, p, pre):
    Q = Q_in @ p[pre + '_fcq_w'].T + p[pre + '_fcq_b']

    def split_heads(t):
        return jnp.concatenate(jnp.split(t, NUM_HEADS, axis=2), axis=0)

    Q_ = split_heads(Q)
    K_ = split_heads(K)
    V_ = split_heads(V)
    score = jnp.einsum('bqd,bkd->bqk', Q_, K_) / math.sqrt(HID)
    A = jax.nn.softmax(score, axis=-1)
    out = Q_ + jnp.einsum('bqk,bkd->bqd', A, V_)
    out = jnp.concatenate(jnp.split(out, NUM_HEADS, axis=0), axis=2)
    out = _ln(out, p[pre + '_ln0_g'], p[pre + '_ln0_b'])
    out = out + jax.nn.relu(out @ p[pre + '_fco_w'].T + p[pre + '_fco_b'])
    out = _ln(out, p[pre + '_ln1_g'], p[pre + '_ln1_b'])
    return out


def kernel(x, edge_index, batch, params):
    p = params
    src = edge_index[0]
    dst = edge_index[1]
    srcp, dstp = _pad_edges(src, dst)
    zn = jnp.zeros((N_NODES,), jnp.float32)
    z16 = jnp.zeros((16,), jnp.float32)
    h = _linear_tc(x, p['enc_w'], p['enc_b'])
    h = _gin_conv(h, srcp, dstp, zn, z16, p, 'c1')
    h = jax.nn.relu(h)
    h = _gin_conv(h, srcp, dstp, zn, z16, p, 'c2')
    xg = h @ p['gmt_lin1_w'].T + p['gmt_lin1_b']
    hK, aKs, aKd, amaxK = _gat_pre(xg, p['gatk_lin_w'], p['gatk_att_src'],
                                   p['gatk_att_dst'])
    hV, aVs, aVd, amaxV = _gat_pre(xg, p['gatv_lin_w'], p['gatv_att_src'],
                                   p['gatv_att_dst'])
    ones_h = jnp.ones((N_NODES, HID), jnp.float32)
    amaxK16 = jnp.broadcast_to(amaxK, (16,))
    amaxV16 = jnp.broadcast_to(amaxV, (16,))
    numK = _sc_edge_pass(hK, aKs, aKd, amaxK16, srcp, dstp)
    hV_, aVs_, aVd_, amaxV16_ = _after(numK, hV, aVs, aVd, amaxV16)
    numV = _sc_edge_pass(hV_, aVs_, aVd_, amaxV16_, srcp, dstp)
    onesK, aKs_, aKd_, amaxK16_ = _after(numV, ones_h, aKs, aKd, amaxK16)
    denKp = _sc_edge_pass(onesK, aKs_, aKd_, amaxK16_, srcp, dstp)
    onesV, aVs2, aVd2, amaxV2 = _after(denKp, ones_h, aVs_, aVd_, amaxV16_)
    denVp = _sc_edge_pass(onesV, aVs2, aVd2, amaxV2, srcp, dstp)
    denK = denKp[0, :N_NODES, 0] + denKp[1, :N_NODES, 0]
    denV = denVp[0, :N_NODES, 0] + denVp[1, :N_NODES, 0]
    K = _gat_post(numK, denK, hK, aKs, aKd, amaxK, p['gatk_bias'])[None]
    V = _gat_post(numV, denV, hV, aVs, aVd, amaxV, p['gatv_bias'])[None]
    S = jnp.broadcast_to(p['pma1_S'], (1, SEEDS1, HID))
    bx = _mab(S, K, V, p, 'mab1')
    K2 = bx @ p['mab2_lk_w'].T + p['mab2_lk_b']
    V2 = bx @ p['mab2_lv_w'].T + p['mab2_lv_b']
    bx = _mab(bx, K2, V2, p, 'mab2')
    K3 = bx @ p['mab3_lk_w'].T + p['mab3_lk_b']
    V3 = bx @ p['mab3_lv_w'].T + p['mab3_lv_b']
    bx = _mab(p['pma2_S'], K3, V3, p, 'mab3')
    out = bx[:, 0, :] @ p['gmt_lin2_w'].T + p['gmt_lin2_b']
    out = out @ p['clf_w'].T + p['clf_b']
    return out


# speedup vs baseline: 10.9719x; 1.4583x over previous
import functools
import math

import jax
import jax.numpy as jnp
from jax import lax
from jax.experimental import pallas as pl
from jax.experimental.pallas import tpu as pltpu
from jax.experimental.pallas import tpu_sc as plsc

N_NODES = 10000
N_EDGES = 320000
IN_CH = 128
HID = 128
OUT_CH = 64
NUM_HEADS = 4
SEEDS1 = 75

_BN_SCALE = 1.0 / math.sqrt(1.0 + 1e-5)


def _linear_body(x_ref, w_ref, b_ref, o_ref):
    o_ref[...] = lax.dot_general(
        x_ref[...], w_ref[...], (((1,), (1,)), ((), ())),
        preferred_element_type=jnp.float32) + b_ref[...]


def _linear_tc(x, w, b, blk=1000):
    n, _ = x.shape
    od, idim = w.shape
    return pl.pallas_call(
        _linear_body,
        grid=(n // blk,),
        in_specs=[
            pl.BlockSpec((blk, idim), lambda i: (i, 0)),
            pl.BlockSpec((od, idim), lambda i: (0, 0)),
            pl.BlockSpec((1, od), lambda i: (0, 0)),
        ],
        out_specs=pl.BlockSpec((blk, od), lambda i: (i, 0)),
        out_shape=jax.ShapeDtypeStruct((n, od), jnp.float32),
    )(x, w, b[None])


_NC = 2
_NS = 16
_NW = _NC * _NS
_CHUNK = 64
_SPT = 16
_NSLAB = 10
_CPT = _SPT * _NSLAB
_N_CHUNKS = _NW * _CPT
_E_PAD = _N_CHUNKS * _CHUNK
_ACC_ROWS = 10240

_SC_MESH = plsc.VectorSubcoreMesh(core_axis_name="c", subcore_axis_name="s")


def _edge_weights(sb, db, asb, adb, amax, j, t):
    s16 = sb.at[j][pl.ds(16 * t, 16)]
    d16 = db.at[j][pl.ds(16 * t, 16)]
    a_s = plsc.load_gather(asb, [s16])
    a_d = plsc.load_gather(adb, [d16])
    al = a_s + a_d
    al = jnp.maximum(al, 0.2 * al)
    c = a_d + amax
    c = jnp.maximum(c, 0.2 * c)
    return jnp.exp(al - c)


def _sc_edge_body(h_hbm, asrc_hbm, adst_hbm, amax_hbm, srcp_hbm, dstp_hbm,
                  out_hbm, *rest, nf, with_den):
    if with_den:
        den_hbm = rest[0]
        rest = rest[1:]
    (srcb0, dstb0, srcb1, dstb1, gbuf0, gbuf1, asb, adb, amaxb,
     wbuf0, wbuf1, dumpb, acc, gsem0, gsem1, ssem0, ssem1,
     isem0, isem1) = rest[:19]
    if with_den:
        dzb, dacc, dsem0, dsem1 = rest[19:]
    cid = lax.axis_index("c")
    sid = lax.axis_index("s")
    wid = cid * _NS + sid
    base = wid * _CPT
    z16 = jnp.zeros((16,), jnp.float32)
    dump16 = jnp.full((16,), N_NODES, jnp.int32)

    def zrow(i, _):
        for t in range(nf):
            gbuf0.at[i][pl.ds(16 * t, 16)] = z16
            gbuf1.at[i][pl.ds(16 * t, 16)] = z16
        return 0

    lax.fori_loop(0, _CHUNK, zrow, 0)
    for t in range(4):
        dumpb[pl.ds(16 * t, 16)] = dump16

    if with_den:
        for t in range(4):
            wbuf0[pl.ds(16 * t, 16)] = z16
            wbuf1[pl.ds(16 * t, 16)] = z16
        for t in range(40):
            dzb[pl.ds(16 * t, 16)] = z16

    def zacc(k, _):
        pltpu.sync_copy(gbuf0, acc.at[pl.ds(sid * 640 + k * 64, 64)])
        return 0

    lax.fori_loop(0, 10, zacc, 0)
    if with_den:
        pltpu.sync_copy(dzb, dacc.at[pl.ds(sid * 640, 640)])

    pltpu.sync_copy(asrc_hbm, asb)
    pltpu.sync_copy(adst_hbm, adb)
    pltpu.sync_copy(amax_hbm, amaxb)
    pltpu.sync_copy(srcp_hbm.at[pl.ds(base, _SPT)], srcb0)
    pltpu.sync_copy(dstp_hbm.at[pl.ds(base, _SPT)], dstb0)
    plsc.subcore_barrier()
    amax = amaxb[...]

    pltpu.async_copy(gbuf0, acc.at[dumpb], ssem0, add=True)
    pltpu.async_copy(gbuf1, acc.at[dumpb], ssem1, add=True)
    if with_den:
        pltpu.async_copy(wbuf0, dacc.at[dumpb], dsem0, add=True)
        pltpu.async_copy(wbuf1, dacc.at[dumpb], dsem1, add=True)
    pltpu.async_copy(h_hbm.at[srcb0.at[0]], gbuf0, gsem0)

    if with_den:
        bufs = ((gbuf0, wbuf0, gsem0, ssem0, dsem0),
                (gbuf1, wbuf1, gsem1, ssem1, dsem1))
    else:
        bufs = ((gbuf0, wbuf0, gsem0, ssem0, None),
                (gbuf1, wbuf1, gsem1, ssem1, None))

    def chunk_work(sb, db, jj, q, nxt):
        gq, wq, gsem, ssem, dsem = bufs[q]
        go, _, gso, _, _ = bufs[1 - q]
        if with_den:
            pltpu.make_async_copy(wq, dacc.at[db.at[jj]], dsem).wait()
        for t in range(4):
            wq[pl.ds(16 * t, 16)] = _edge_weights(sb, db, asb, adb, amax,
                                                  jj, t)
        pltpu.make_async_copy(h_hbm.at[sb.at[jj]], gq, gsem).wait()
        if nxt is not None:
            nsb, njj = nxt
            pltpu.async_copy(h_hbm.at[nsb.at[njj]], go, gso)
        pltpu.make_async_copy(gq, acc.at[db.at[jj]], ssem).wait()

        def scale(e, _):
            ev = jnp.broadcast_to(e, (16,)).astype(jnp.int32)
            we = plsc.load_gather(wq, [ev])
            for t in range(nf):
                gq.at[e][pl.ds(16 * t, 16)] = (
                    gq.at[e][pl.ds(16 * t, 16)] * we)
            return 0

        lax.fori_loop(0, _CHUNK, scale, 0)
        pltpu.async_copy(gq, acc.at[db.at[jj]], ssem, add=True)
        if with_den:
            pltpu.async_copy(wq, dacc.at[db.at[jj]], dsem, add=True)

    def pair(p, _):
        a0 = base + 2 * p * _SPT
        b0 = a0 + _SPT
        for j in range(_SPT):
            if j == 3:
                pltpu.async_copy(srcp_hbm.at[pl.ds(b0, _SPT)], srcb1, isem1)
                pltpu.async_copy(dstp_hbm.at[pl.ds(b0, _SPT)], dstb1, isem1)
            if j == _SPT - 1:
                pltpu.make_async_copy(srcp_hbm.at[pl.ds(b0, _SPT)], srcb1,
                                      isem1).wait()
                pltpu.make_async_copy(dstp_hbm.at[pl.ds(b0, _SPT)], dstb1,
                                      isem1).wait()
                nxt = (srcb1, 0)
            else:
                nxt = (srcb0, j + 1)
            chunk_work(srcb0, dstb0, j, j % 2, nxt)
        for j in range(_SPT):
            if j == 3:
                @pl.when(p < _NSLAB // 2 - 1)
                def _():
                    nb = base + (2 * p + 2) * _SPT
                    pltpu.async_copy(srcp_hbm.at[pl.ds(nb, _SPT)], srcb0,
                                     isem0)
                    pltpu.async_copy(dstp_hbm.at[pl.ds(nb, _SPT)], dstb0,
                                     isem0)
            if j == _SPT - 1:
                @pl.when(p < _NSLAB // 2 - 1)
                def _():
                    nb = base + (2 * p + 2) * _SPT
                    pltpu.make_async_copy(srcp_hbm.at[pl.ds(nb, _SPT)],
                                          srcb0, isem0).wait()
                    pltpu.make_async_copy(dstp_hbm.at[pl.ds(nb, _SPT)],
                                          dstb0, isem0).wait()
                    chunk_work(srcb1, dstb1, _SPT - 1, (_SPT - 1) % 2,
                               (srcb0, 0))

                @pl.when(p >= _NSLAB // 2 - 1)
                def _():
                    chunk_work(srcb1, dstb1, _SPT - 1, (_SPT - 1) % 2, None)
            else:
                chunk_work(srcb1, dstb1, j, j % 2, (srcb1, j + 1))
        return 0

    lax.fori_loop(0, _NSLAB // 2, pair, 0)

    pltpu.make_async_copy(gbuf0, acc.at[dumpb], ssem0).wait()
    pltpu.make_async_copy(gbuf1, acc.at[dumpb], ssem1).wait()
    if with_den:
        pltpu.make_async_copy(wbuf0, dacc.at[dumpb], dsem0).wait()
        pltpu.make_async_copy(wbuf1, dacc.at[dumpb], dsem1).wait()
    plsc.subcore_barrier()
    pltpu.sync_copy(acc.at[pl.ds(sid * 640, 640)],
                    out_hbm.at[cid, pl.ds(sid * 640, 640)])
    if with_den:
        pltpu.sync_copy(dacc.at[pl.ds(sid * 640, 640)],
                        den_hbm.at[cid, pl.ds(sid * 640, 640)])


def _sc_edge_pass(h, a_src, a_dst, amax, srcp, dstp, with_den=False):
    width = h.shape[1]
    out_type = jax.ShapeDtypeStruct((_NC, _ACC_ROWS, width), jnp.float32)
    if with_den:
        out_type = [out_type,
                    jax.ShapeDtypeStruct((_NC, _ACC_ROWS), jnp.float32)]
    scratch = [
        pltpu.VMEM((_SPT, _CHUNK), jnp.int32),
        pltpu.VMEM((_SPT, _CHUNK), jnp.int32),
        pltpu.VMEM((_SPT, _CHUNK), jnp.int32),
        pltpu.VMEM((_SPT, _CHUNK), jnp.int32),
        pltpu.VMEM((_CHUNK, width), jnp.float32),
        pltpu.VMEM((_CHUNK, width), jnp.float32),
        pltpu.VMEM((N_NODES,), jnp.float32),
        pltpu.VMEM((N_NODES,), jnp.float32),
        pltpu.VMEM((16,), jnp.float32),
        pltpu.VMEM((_CHUNK,), jnp.float32),
        pltpu.VMEM((_CHUNK,), jnp.float32),
        pltpu.VMEM((_CHUNK,), jnp.int32),
        pltpu.MemorySpace.VMEM_SHARED((_ACC_ROWS, width), jnp.float32),
        pltpu.SemaphoreType.DMA,
        pltpu.SemaphoreType.DMA,
        pltpu.SemaphoreType.DMA,
        pltpu.SemaphoreType.DMA,
        pltpu.SemaphoreType.DMA,
        pltpu.SemaphoreType.DMA,
    ]
    if with_den:
        scratch += [
            pltpu.VMEM((640,), jnp.float32),
            pltpu.MemorySpace.VMEM_SHARED((_ACC_ROWS,), jnp.float32),
            pltpu.SemaphoreType.DMA,
            pltpu.SemaphoreType.DMA,
        ]
    f = pl.kernel(
        functools.partial(_sc_edge_body, nf=width // 16, with_den=with_den),
        out_type=out_type,
        mesh=_SC_MESH,
        scratch_types=scratch,
        compiler_params=pltpu.CompilerParams(needs_layout_passes=False),
    )
    return f(h, a_src, a_dst, amax, srcp, dstp)


def _pad_edges(src, dst):
    npad = _E_PAD - N_EDGES
    srcp = jnp.concatenate(
        [src, jnp.zeros((npad,), jnp.int32)]).reshape(_N_CHUNKS, _CHUNK)
    dstp = jnp.concatenate(
        [dst, jnp.full((npad,), N_NODES, jnp.int32)]).reshape(
            _N_CHUNKS, _CHUNK)
    return srcp, dstp


def _bn_eval(x, g, b):
    return x * (_BN_SCALE * g) + b


def _ln(x, g, b):
    m = jnp.mean(x, axis=-1, keepdims=True)
    v = jnp.mean((x - m) ** 2, axis=-1, keepdims=True)
    return (x - m) / jnp.sqrt(v + 1e-5) * g + b


def _mlp(x, p, pre):
    x = x @ p[pre + '_l0_w'].T + p[pre + '_l0_b']
    x = _bn_eval(x, p[pre + '_bn0_g'], p[pre + '_bn0_b'])
    x = jax.nn.relu(x)
    x = x @ p[pre + '_l1_w'].T + p[pre + '_l1_b']
    x = _bn_eval(x, p[pre + '_bn1_g'], p[pre + '_bn1_b'])
    return x


_Z_NODE = None


def _gin_conv(x, srcp, dstp, zn, z16, p, pre):
    parts = _sc_edge_pass(x, zn, zn, z16, srcp, dstp)
    return _mlp(x + parts[0, :N_NODES] + parts[1, :N_NODES], p, pre)


def _after(dep, *xs):
    return lax.optimization_barrier((dep, xs))[1]


def _gat_pre(x, w, att_src, att_dst):
    h = x @ w.T
    a_src = jnp.sum(h * att_src, axis=-1)
    a_dst = jnp.sum(h * att_dst, axis=-1)
    amax = jnp.max(a_src)
    return h, a_src, a_dst, amax


def _gat_post(num, den, h, a_src, a_dst, amax, bias):
    al = jax.nn.leaky_relu(a_src + a_dst, 0.2)
    c = jax.nn.leaky_relu(a_dst + amax, 0.2)
    wl = jnp.exp(al - c)
    ntot = num + wl[:, None] * h
    dtot = den + wl
    return ntot / (dtot + 1e-16)[:, None] + bias


def _mab(Q_in, K, V, p, pre):
    Q = Q_in @ p[pre + '_fcq_w'].T + p[pre + '_fcq_b']

    def split_heads(t):
        return jnp.concatenate(jnp.split(t, NUM_HEADS, axis=2), axis=0)

    Q_ = split_heads(Q)
    K_ = split_heads(K)
    V_ = split_heads(V)
    score = jnp.einsum('bqd,bkd->bqk', Q_, K_) / math.sqrt(HID)
    A = jax.nn.softmax(score, axis=-1)
    out = Q_ + jnp.einsum('bqk,bkd->bqd', A, V_)
    out = jnp.concatenate(jnp.split(out, NUM_HEADS, axis=0), axis=2)
    out = _ln(out, p[pre + '_ln0_g'], p[pre + '_ln0_b'])
    out = out + jax.nn.relu(out @ p[pre + '_fco_w'].T + p[pre + '_fco_b'])
    out = _ln(out, p[pre + '_ln1_g'], p[pre + '_ln1_b'])
    return out


def kernel(x, edge_index, batch, params):
    p = params
    src = edge_index[0]
    dst = edge_index[1]
    srcp, dstp = _pad_edges(src, dst)
    zn = jnp.zeros((N_NODES,), jnp.float32)
    z16 = jnp.zeros((16,), jnp.float32)
    h = _linear_tc(x, p['enc_w'], p['enc_b'])
    h = _gin_conv(h, srcp, dstp, zn, z16, p, 'c1')
    h = jax.nn.relu(h)
    h = _gin_conv(h, srcp, dstp, zn, z16, p, 'c2')
    xg = h @ p['gmt_lin1_w'].T + p['gmt_lin1_b']
    hK, aKs, aKd, amaxK = _gat_pre(xg, p['gatk_lin_w'], p['gatk_att_src'],
                                   p['gatk_att_dst'])
    hV, aVs, aVd, amaxV = _gat_pre(xg, p['gatv_lin_w'], p['gatv_att_src'],
                                   p['gatv_att_dst'])
    amaxK16 = jnp.broadcast_to(amaxK, (16,))
    amaxV16 = jnp.broadcast_to(amaxV, (16,))
    numK, wsK = _sc_edge_pass(hK, aKs, aKd, amaxK16, srcp, dstp,
                              with_den=True)
    hV_, aVs_, aVd_, amaxV16_ = _after((numK, wsK), hV, aVs, aVd, amaxV16)
    numV, wsV = _sc_edge_pass(hV_, aVs_, aVd_, amaxV16_, srcp, dstp,
                              with_den=True)
    nK = numK[0, :N_NODES] + numK[1, :N_NODES]
    nV = numV[0, :N_NODES] + numV[1, :N_NODES]
    denK = wsK[0, :N_NODES] + wsK[1, :N_NODES]
    denV = wsV[0, :N_NODES] + wsV[1, :N_NODES]
    K = _gat_post(nK, denK, hK, aKs, aKd, amaxK, p['gatk_bias'])[None]
    V = _gat_post(nV, denV, hV, aVs, aVd, amaxV, p['gatv_bias'])[None]
    S = jnp.broadcast_to(p['pma1_S'], (1, SEEDS1, HID))
    bx = _mab(S, K, V, p, 'mab1')
    K2 = bx @ p['mab2_lk_w'].T + p['mab2_lk_b']
    V2 = bx @ p['mab2_lv_w'].T + p['mab2_lv_b']
    bx = _mab(bx, K2, V2, p, 'mab2')
    K3 = bx @ p['mab3_lk_w'].T + p['mab3_lk_b']
    V3 = bx @ p['mab3_lv_w'].T + p['mab3_lv_b']
    bx = _mab(p['pma2_S'], K3, V3, p, 'mab3')
    out = bx[:, 0, :] @ p['gmt_lin2_w'].T + p['gmt_lin2_b']
    out = out @ p['clf_w'].T + p['clf_b']
    return out


# speedup vs baseline: 11.0062x; 1.0031x over previous
import functools
import math

import jax
import jax.numpy as jnp
from jax import lax
from jax.experimental import pallas as pl
from jax.experimental.pallas import tpu as pltpu
from jax.experimental.pallas import tpu_sc as plsc

N_NODES = 10000
N_EDGES = 320000
IN_CH = 128
HID = 128
OUT_CH = 64
NUM_HEADS = 4
SEEDS1 = 75

_BN_SCALE = 1.0 / math.sqrt(1.0 + 1e-5)


def _linear_body(x_ref, w_ref, b_ref, o_ref):
    o_ref[...] = lax.dot_general(
        x_ref[...], w_ref[...], (((1,), (1,)), ((), ())),
        preferred_element_type=jnp.float32) + b_ref[...]


def _linear_tc(x, w, b, blk=1000):
    n, _ = x.shape
    od, idim = w.shape
    return pl.pallas_call(
        _linear_body,
        grid=(n // blk,),
        in_specs=[
            pl.BlockSpec((blk, idim), lambda i: (i, 0)),
            pl.BlockSpec((od, idim), lambda i: (0, 0)),
            pl.BlockSpec((1, od), lambda i: (0, 0)),
        ],
        out_specs=pl.BlockSpec((blk, od), lambda i: (i, 0)),
        out_shape=jax.ShapeDtypeStruct((n, od), jnp.float32),
    )(x, w, b[None])


_NC = 2
_NS = 16
_NW = _NC * _NS
_CHUNK = 64
_SPT = 16
_NSLAB = 10
_CPT = _SPT * _NSLAB
_N_CHUNKS = _NW * _CPT
_E_PAD = _N_CHUNKS * _CHUNK
_ACC_ROWS = 10240

_SC_MESH = plsc.VectorSubcoreMesh(core_axis_name="c", subcore_axis_name="s")


def _edge_weights(sb, db, asb, adb, amax, j, t):
    s16 = sb.at[j][pl.ds(16 * t, 16)]
    d16 = db.at[j][pl.ds(16 * t, 16)]
    a_s = plsc.load_gather(asb, [s16])
    a_d = plsc.load_gather(adb, [d16])
    al = a_s + a_d
    al = jnp.maximum(al, 0.2 * al)
    c = a_d + amax
    c = jnp.maximum(c, 0.2 * c)
    return jnp.exp(al - c)


def _sc_edge_body(h_hbm, asrc_hbm, adst_hbm, amax_hbm, srcp_hbm, dstp_hbm,
                  out_hbm, *rest, nf, with_den):
    if with_den:
        den_hbm = rest[0]
        rest = rest[1:]
    (srcb0, dstb0, srcb1, dstb1, gbuf0, gbuf1, asb, adb, amaxb,
     wbuf0, wbuf1, dumpb, acc, gsem0, gsem1, ssem0, ssem1,
     isem0, isem1) = rest[:19]
    if with_den:
        dzb, dacc, dsem0, dsem1 = rest[19:]
    cid = lax.axis_index("c")
    sid = lax.axis_index("s")
    wid = cid * _NS + sid
    base = wid * _CPT
    z16 = jnp.zeros((16,), jnp.float32)
    lane16 = lax.iota(jnp.int32, 16)

    def zrow(i, _):
        for t in range(nf):
            gbuf0.at[i][pl.ds(16 * t, 16)] = z16
            gbuf1.at[i][pl.ds(16 * t, 16)] = z16
        return 0

    lax.fori_loop(0, _CHUNK, zrow, 0)
    for t in range(4):
        dumpb[pl.ds(16 * t, 16)] = N_NODES + 16 * t + lane16

    if with_den:
        for t in range(4):
            wbuf0[pl.ds(16 * t, 16)] = z16
            wbuf1[pl.ds(16 * t, 16)] = z16
        for t in range(40):
            dzb[pl.ds(16 * t, 16)] = z16

    def zacc(k, _):
        pltpu.sync_copy(gbuf0, acc.at[pl.ds(sid * 640 + k * 64, 64)])
        return 0

    lax.fori_loop(0, 10, zacc, 0)
    if with_den:
        pltpu.sync_copy(dzb, dacc.at[pl.ds(sid * 640, 640)])

    pltpu.sync_copy(asrc_hbm, asb)
    pltpu.sync_copy(adst_hbm, adb)
    pltpu.sync_copy(amax_hbm, amaxb)
    pltpu.sync_copy(srcp_hbm.at[pl.ds(base, _SPT)], srcb0)
    pltpu.sync_copy(dstp_hbm.at[pl.ds(base, _SPT)], dstb0)
    plsc.subcore_barrier()
    amax = amaxb[...]

    pltpu.async_copy(gbuf0, acc.at[dumpb], ssem0, add=True)
    pltpu.async_copy(gbuf1, acc.at[dumpb], ssem1, add=True)
    if with_den:
        pltpu.async_copy(wbuf0, dacc.at[dumpb], dsem0, add=True)
        pltpu.async_copy(wbuf1, dacc.at[dumpb], dsem1, add=True)
    pltpu.async_copy(h_hbm.at[srcb0.at[0]], gbuf0, gsem0)

    if with_den:
        bufs = ((gbuf0, wbuf0, gsem0, ssem0, dsem0),
                (gbuf1, wbuf1, gsem1, ssem1, dsem1))
    else:
        bufs = ((gbuf0, wbuf0, gsem0, ssem0, None),
                (gbuf1, wbuf1, gsem1, ssem1, None))

    def chunk_work(sb, db, jj, q, nxt):
        gq, wq, gsem, ssem, dsem = bufs[q]
        go, _, gso, _, _ = bufs[1 - q]
        if with_den:
            pltpu.make_async_copy(wq, dacc.at[db.at[jj]], dsem).wait()
        for t in range(4):
            wq[pl.ds(16 * t, 16)] = _edge_weights(sb, db, asb, adb, amax,
                                                  jj, t)
        pltpu.make_async_copy(h_hbm.at[sb.at[jj]], gq, gsem).wait()
        if nxt is not None:
            nsb, njj = nxt
            pltpu.async_copy(h_hbm.at[nsb.at[njj]], go, gso)
        pltpu.make_async_copy(gq, acc.at[db.at[jj]], ssem).wait()

        def scale(e, _):
            ev = jnp.broadcast_to(e, (16,)).astype(jnp.int32)
            we = plsc.load_gather(wq, [ev])
            for t in range(nf):
                gq.at[e][pl.ds(16 * t, 16)] = (
                    gq.at[e][pl.ds(16 * t, 16)] * we)
            return 0

        lax.fori_loop(0, _CHUNK, scale, 0)
        pltpu.async_copy(gq, acc.at[db.at[jj]], ssem, add=True)
        if with_den:
            pltpu.async_copy(wq, dacc.at[db.at[jj]], dsem, add=True)

    def pair(p, _):
        a0 = base + 2 * p * _SPT
        b0 = a0 + _SPT
        for j in range(_SPT):
            if j == 3:
                pltpu.async_copy(srcp_hbm.at[pl.ds(b0, _SPT)], srcb1, isem1)
                pltpu.async_copy(dstp_hbm.at[pl.ds(b0, _SPT)], dstb1, isem1)
            if j == _SPT - 1:
                pltpu.make_async_copy(srcp_hbm.at[pl.ds(b0, _SPT)], srcb1,
                                      isem1).wait()
                pltpu.make_async_copy(dstp_hbm.at[pl.ds(b0, _SPT)], dstb1,
                                      isem1).wait()
                nxt = (srcb1, 0)
            else:
                nxt = (srcb0, j + 1)
            chunk_work(srcb0, dstb0, j, j % 2, nxt)
        for j in range(_SPT):
            if j == 3:
                @pl.when(p < _NSLAB // 2 - 1)
                def _():
                    nb = base + (2 * p + 2) * _SPT
                    pltpu.async_copy(srcp_hbm.at[pl.ds(nb, _SPT)], srcb0,
                                     isem0)
                    pltpu.async_copy(dstp_hbm.at[pl.ds(nb, _SPT)], dstb0,
                                     isem0)
            if j == _SPT - 1:
                @pl.when(p < _NSLAB // 2 - 1)
                def _():
                    nb = base + (2 * p + 2) * _SPT
                    pltpu.make_async_copy(srcp_hbm.at[pl.ds(nb, _SPT)],
                                          srcb0, isem0).wait()
                    pltpu.make_async_copy(dstp_hbm.at[pl.ds(nb, _SPT)],
                                          dstb0, isem0).wait()
                    chunk_work(srcb1, dstb1, _SPT - 1, (_SPT - 1) % 2,
                               (srcb0, 0))

                @pl.when(p >= _NSLAB // 2 - 1)
                def _():
                    chunk_work(srcb1, dstb1, _SPT - 1, (_SPT - 1) % 2, None)
            else:
                chunk_work(srcb1, dstb1, j, j % 2, (srcb1, j + 1))
        return 0

    lax.fori_loop(0, _NSLAB // 2, pair, 0)

    pltpu.make_async_copy(gbuf0, acc.at[dumpb], ssem0).wait()
    pltpu.make_async_copy(gbuf1, acc.at[dumpb], ssem1).wait()
    if with_den:
        pltpu.make_async_copy(wbuf0, dacc.at[dumpb], dsem0).wait()
        pltpu.make_async_copy(wbuf1, dacc.at[dumpb], dsem1).wait()
    plsc.subcore_barrier()
    pltpu.sync_copy(acc.at[pl.ds(sid * 640, 640)],
                    out_hbm.at[cid, pl.ds(sid * 640, 640)])
    if with_den:
        pltpu.sync_copy(dacc.at[pl.ds(sid * 640, 640)],
                        den_hbm.at[cid, pl.ds(sid * 640, 640)])


def _sc_edge_pass(h, a_src, a_dst, amax, srcp, dstp, with_den=False):
    width = h.shape[1]
    out_type = jax.ShapeDtypeStruct((_NC, _ACC_ROWS, width), jnp.float32)
    if with_den:
        out_type = [out_type,
                    jax.ShapeDtypeStruct((_NC, _ACC_ROWS), jnp.float32)]
    scratch = [
        pltpu.VMEM((_SPT, _CHUNK), jnp.int32),
        pltpu.VMEM((_SPT, _CHUNK), jnp.int32),
        pltpu.VMEM((_SPT, _CHUNK), jnp.int32),
        pltpu.VMEM((_SPT, _CHUNK), jnp.int32),
        pltpu.VMEM((_CHUNK, width), jnp.float32),
        pltpu.VMEM((_CHUNK, width), jnp.float32),
        pltpu.VMEM((N_NODES,), jnp.float32),
        pltpu.VMEM((N_NODES,), jnp.float32),
        pltpu.VMEM((16,), jnp.float32),
        pltpu.VMEM((_CHUNK,), jnp.float32),
        pltpu.VMEM((_CHUNK,), jnp.float32),
        pltpu.VMEM((_CHUNK,), jnp.int32),
        pltpu.MemorySpace.VMEM_SHARED((_ACC_ROWS, width), jnp.float32),
        pltpu.SemaphoreType.DMA,
        pltpu.SemaphoreType.DMA,
        pltpu.SemaphoreType.DMA,
        pltpu.SemaphoreType.DMA,
        pltpu.SemaphoreType.DMA,
        pltpu.SemaphoreType.DMA,
    ]
    if with_den:
        scratch += [
            pltpu.VMEM((640,), jnp.float32),
            pltpu.MemorySpace.VMEM_SHARED((_ACC_ROWS,), jnp.float32),
            pltpu.SemaphoreType.DMA,
            pltpu.SemaphoreType.DMA,
        ]
    f = pl.kernel(
        functools.partial(_sc_edge_body, nf=width // 16, with_den=with_den),
        out_type=out_type,
        mesh=_SC_MESH,
        scratch_types=scratch,
        compiler_params=pltpu.CompilerParams(needs_layout_passes=False),
    )
    return f(h, a_src, a_dst, amax, srcp, dstp)


def _pad_edges(src, dst):
    npad = _E_PAD - N_EDGES
    pad_dst = N_NODES + (jnp.arange(npad, dtype=jnp.int32)
                         % (_ACC_ROWS - N_NODES))
    srcp = jnp.concatenate(
        [src, jnp.zeros((npad,), jnp.int32)]).reshape(_N_CHUNKS, _CHUNK)
    dstp = jnp.concatenate([dst, pad_dst]).reshape(_N_CHUNKS, _CHUNK)
    return srcp, dstp


def _bn_eval(x, g, b):
    return x * (_BN_SCALE * g) + b


def _ln(x, g, b):
    m = jnp.mean(x, axis=-1, keepdims=True)
    v = jnp.mean((x - m) ** 2, axis=-1, keepdims=True)
    return (x - m) / jnp.sqrt(v + 1e-5) * g + b


def _mlp(x, p, pre):
    x = x @ p[pre + '_l0_w'].T + p[pre + '_l0_b']
    x = _bn_eval(x, p[pre + '_bn0_g'], p[pre + '_bn0_b'])
    x = jax.nn.relu(x)
    x = x @ p[pre + '_l1_w'].T + p[pre + '_l1_b']
    x = _bn_eval(x, p[pre + '_bn1_g'], p[pre + '_bn1_b'])
    return x


_Z_NODE = None


def _gin_conv(x, srcp, dstp, zn, z16, p, pre):
    parts = _sc_edge_pass(x, zn, zn, z16, srcp, dstp)
    return _mlp(x + parts[0, :N_NODES] + parts[1, :N_NODES], p, pre)


def _after(dep, *xs):
    return lax.optimization_barrier((dep, xs))[1]


def _gat_pre(x, w, att_src, att_dst):
    h = x @ w.T
    a_src = jnp.sum(h * att_src, axis=-1)
    a_dst = jnp.sum(h * att_dst, axis=-1)
    amax = jnp.max(a_src)
    return h, a_src, a_dst, amax


def _gat_post(num, den, h, a_src, a_dst, amax, bias):
    al = jax.nn.leaky_relu(a_src + a_dst, 0.2)
    c = jax.nn.leaky_relu(a_dst + amax, 0.2)
    wl = jnp.exp(al - c)
    ntot = num + wl[:, None] * h
    dtot = den + wl
    return ntot / (dtot + 1e-16)[:, None] + bias


def _mab(Q_in, K, V, p, pre):
    Q = Q_in @ p[pre + '_fcq_w'].T + p[pre + '_fcq_b']

    def split_heads(t):
        return jnp.concatenate(jnp.split(t, NUM_HEADS, axis=2), axis=0)

    Q_ = split_heads(Q)
    K_ = split_heads(K)
    V_ = split_heads(V)
    score = jnp.einsum('bqd,bkd->bqk', Q_, K_) / math.sqrt(HID)
    A = jax.nn.softmax(score, axis=-1)
    out = Q_ + jnp.einsum('bqk,bkd->bqd', A, V_)
    out = jnp.concatenate(jnp.split(out, NUM_HEADS, axis=0), axis=2)
    out = _ln(out, p[pre + '_ln0_g'], p[pre + '_ln0_b'])
    out = out + jax.nn.relu(out @ p[pre + '_fco_w'].T + p[pre + '_fco_b'])
    out = _ln(out, p[pre + '_ln1_g'], p[pre + '_ln1_b'])
    return out


def kernel(x, edge_index, batch, params):
    p = params
    src = edge_index[0]
    dst = edge_index[1]
    srcp, dstp = _pad_edges(src, dst)
    zn = jnp.zeros((N_NODES,), jnp.float32)
    z16 = jnp.zeros((16,), jnp.float32)
    h = _linear_tc(x, p['enc_w'], p['enc_b'])
    h = _gin_conv(h, srcp, dstp, zn, z16, p, 'c1')
    h = jax.nn.relu(h)
    h = _gin_conv(h, srcp, dstp, zn, z16, p, 'c2')
    xg = h @ p['gmt_lin1_w'].T + p['gmt_lin1_b']
    hK, aKs, aKd, amaxK = _gat_pre(xg, p['gatk_lin_w'], p['gatk_att_src'],
                                   p['gatk_att_dst'])
    hV, aVs, aVd, amaxV = _gat_pre(xg, p['gatv_lin_w'], p['gatv_att_src'],
                                   p['gatv_att_dst'])
    amaxK16 = jnp.broadcast_to(amaxK, (16,))
    amaxV16 = jnp.broadcast_to(amaxV, (16,))
    numK, wsK = _sc_edge_pass(hK, aKs, aKd, amaxK16, srcp, dstp,
                              with_den=True)
    hV_, aVs_, aVd_, amaxV16_ = _after((numK, wsK), hV, aVs, aVd, amaxV16)
    numV, wsV = _sc_edge_pass(hV_, aVs_, aVd_, amaxV16_, srcp, dstp,
                              with_den=True)
    nK = numK[0, :N_NODES] + numK[1, :N_NODES]
    nV = numV[0, :N_NODES] + numV[1, :N_NODES]
    denK = wsK[0, :N_NODES] + wsK[1, :N_NODES]
    denV = wsV[0, :N_NODES] + wsV[1, :N_NODES]
    K = _gat_post(nK, denK, hK, aKs, aKd, amaxK, p['gatk_bias'])[None]
    V = _gat_post(nV, denV, hV, aVs, aVd, amaxV, p['gatv_bias'])[None]
    S = jnp.broadcast_to(p['pma1_S'], (1, SEEDS1, HID))
    bx = _mab(S, K, V, p, 'mab1')
    K2 = bx @ p['mab2_lk_w'].T + p['mab2_lk_b']
    V2 = bx @ p['mab2_lv_w'].T + p['mab2_lv_b']
    bx = _mab(bx, K2, V2, p, 'mab2')
    K3 = bx @ p['mab3_lk_w'].T + p['mab3_lk_b']
    V3 = bx @ p['mab3_lv_w'].T + p['mab3_lv_b']
    bx = _mab(p['pma2_S'], K3, V3, p, 'mab3')
    out = bx[:, 0, :] @ p['gmt_lin2_w'].T + p['gmt_lin2_b']
    out = out @ p['clf_w'].T + p['clf_b']
    return out


# speedup vs baseline: 11.0928x; 1.0079x over previous
import functools
import math

import jax
import jax.numpy as jnp
from jax import lax
from jax.experimental import pallas as pl
from jax.experimental.pallas import tpu as pltpu
from jax.experimental.pallas import tpu_sc as plsc

N_NODES = 10000
N_EDGES = 320000
IN_CH = 128
HID = 128
OUT_CH = 64
NUM_HEADS = 4
SEEDS1 = 75

_BN_SCALE = 1.0 / math.sqrt(1.0 + 1e-5)


def _linear_body(x_ref, w_ref, b_ref, o_ref):
    o_ref[...] = lax.dot_general(
        x_ref[...], w_ref[...], (((1,), (1,)), ((), ())),
        preferred_element_type=jnp.float32) + b_ref[...]


def _linear_tc(x, w, b, blk=1000):
    n, _ = x.shape
    od, idim = w.shape
    return pl.pallas_call(
        _linear_body,
        grid=(n // blk,),
        in_specs=[
            pl.BlockSpec((blk, idim), lambda i: (i, 0)),
            pl.BlockSpec((od, idim), lambda i: (0, 0)),
            pl.BlockSpec((1, od), lambda i: (0, 0)),
        ],
        out_specs=pl.BlockSpec((blk, od), lambda i: (i, 0)),
        out_shape=jax.ShapeDtypeStruct((n, od), jnp.float32),
    )(x, w, b[None])


_NC = 2
_NS = 16
_NW = _NC * _NS
_CHUNK = 64
_SPT = 16
_NSLAB = 10
_CPT = _SPT * _NSLAB
_N_CHUNKS = _NW * _CPT
_E_PAD = _N_CHUNKS * _CHUNK
_ACC_ROWS = 10240

_SC_MESH = plsc.VectorSubcoreMesh(core_axis_name="c", subcore_axis_name="s")


def _edge_weights(sb, db, asb, adb, amax, j, t):
    s16 = sb.at[j][pl.ds(16 * t, 16)]
    d16 = db.at[j][pl.ds(16 * t, 16)]
    a_s = plsc.load_gather(asb, [s16])
    a_d = plsc.load_gather(adb, [d16])
    al = a_s + a_d
    al = jnp.maximum(al, 0.2 * al)
    c = a_d + amax
    c = jnp.maximum(c, 0.2 * c)
    return jnp.exp(al - c)


def _sc_edge_body(h_hbm, asrc_hbm, adst_hbm, amax_hbm, srcp_hbm, dstp_hbm,
                  out_hbm, *rest, nf, with_den, weighted):
    if with_den:
        den_hbm = rest[0]
        rest = rest[1:]
    (srcb0, dstb0, srcb1, dstb1, gbuf0, gbuf1, asb, adb, amaxb,
     wbuf0, wbuf1, dumpb, acc, gsem0, gsem1, ssem0, ssem1,
     isem0, isem1) = rest[:19]
    if with_den:
        dzb, dacc, dsem0, dsem1 = rest[19:]
    cid = lax.axis_index("c")
    sid = lax.axis_index("s")
    wid = cid * _NS + sid
    base = wid * _CPT
    z16 = jnp.zeros((16,), jnp.float32)
    lane16 = lax.iota(jnp.int32, 16)

    def zrow(i, _):
        for t in range(nf):
            gbuf0.at[i][pl.ds(16 * t, 16)] = z16
            gbuf1.at[i][pl.ds(16 * t, 16)] = z16
        return 0

    lax.fori_loop(0, _CHUNK, zrow, 0)
    for t in range(4):
        dumpb[pl.ds(16 * t, 16)] = N_NODES + 16 * t + lane16

    if with_den:
        for t in range(4):
            wbuf0[pl.ds(16 * t, 16)] = z16
            wbuf1[pl.ds(16 * t, 16)] = z16
        for t in range(40):
            dzb[pl.ds(16 * t, 16)] = z16

    def zacc(k, _):
        pltpu.sync_copy(gbuf0, acc.at[pl.ds(sid * 640 + k * 64, 64)])
        return 0

    lax.fori_loop(0, 10, zacc, 0)
    if with_den:
        pltpu.sync_copy(dzb, dacc.at[pl.ds(sid * 640, 640)])

    if weighted:
        pltpu.sync_copy(asrc_hbm, asb)
        pltpu.sync_copy(adst_hbm, adb)
        pltpu.sync_copy(amax_hbm, amaxb)
    pltpu.sync_copy(srcp_hbm.at[pl.ds(base, _SPT)], srcb0)
    pltpu.sync_copy(dstp_hbm.at[pl.ds(base, _SPT)], dstb0)
    plsc.subcore_barrier()
    amax = amaxb[...] if weighted else None

    pltpu.async_copy(gbuf0, acc.at[dumpb], ssem0, add=True)
    pltpu.async_copy(gbuf1, acc.at[dumpb], ssem1, add=True)
    if with_den:
        pltpu.async_copy(wbuf0, dacc.at[dumpb], dsem0, add=True)
        pltpu.async_copy(wbuf1, dacc.at[dumpb], dsem1, add=True)
    pltpu.async_copy(h_hbm.at[srcb0.at[0]], gbuf0, gsem0)

    if with_den:
        bufs = ((gbuf0, wbuf0, gsem0, ssem0, dsem0),
                (gbuf1, wbuf1, gsem1, ssem1, dsem1))
    else:
        bufs = ((gbuf0, wbuf0, gsem0, ssem0, None),
                (gbuf1, wbuf1, gsem1, ssem1, None))

    def chunk_work(sb, db, jj, q, nxt):
        gq, wq, gsem, ssem, dsem = bufs[q]
        go, _, gso, _, _ = bufs[1 - q]
        if with_den:
            pltpu.make_async_copy(wq, dacc.at[db.at[jj]], dsem).wait()
        if weighted:
            for t in range(4):
                wq[pl.ds(16 * t, 16)] = _edge_weights(sb, db, asb, adb,
                                                      amax, jj, t)
        pltpu.make_async_copy(h_hbm.at[sb.at[jj]], gq, gsem).wait()
        if nxt is not None:
            nsb, njj = nxt
            pltpu.async_copy(h_hbm.at[nsb.at[njj]], go, gso)
        pltpu.make_async_copy(gq, acc.at[db.at[jj]], ssem).wait()

        def scale(e, _):
            ev = jnp.broadcast_to(e, (16,)).astype(jnp.int32)
            we = plsc.load_gather(wq, [ev])
            for t in range(nf):
                gq.at[e][pl.ds(16 * t, 16)] = (
                    gq.at[e][pl.ds(16 * t, 16)] * we)
            return 0

        if weighted:
            lax.fori_loop(0, _CHUNK, scale, 0)
        pltpu.async_copy(gq, acc.at[db.at[jj]], ssem, add=True)
        if with_den:
            pltpu.async_copy(wq, dacc.at[db.at[jj]], dsem, add=True)

    def pair(p, _):
        a0 = base + 2 * p * _SPT
        b0 = a0 + _SPT
        for j in range(_SPT):
            if j == 3:
                pltpu.async_copy(srcp_hbm.at[pl.ds(b0, _SPT)], srcb1, isem1)
                pltpu.async_copy(dstp_hbm.at[pl.ds(b0, _SPT)], dstb1, isem1)
            if j == _SPT - 1:
                pltpu.make_async_copy(srcp_hbm.at[pl.ds(b0, _SPT)], srcb1,
                                      isem1).wait()
                pltpu.make_async_copy(dstp_hbm.at[pl.ds(b0, _SPT)], dstb1,
                                      isem1).wait()
                nxt = (srcb1, 0)
            else:
                nxt = (srcb0, j + 1)
            chunk_work(srcb0, dstb0, j, j % 2, nxt)
        for j in range(_SPT):
            if j == 3:
                @pl.when(p < _NSLAB // 2 - 1)
                def _():
                    nb = base + (2 * p + 2) * _SPT
                    pltpu.async_copy(srcp_hbm.at[pl.ds(nb, _SPT)], srcb0,
                                     isem0)
                    pltpu.async_copy(dstp_hbm.at[pl.ds(nb, _SPT)], dstb0,
                                     isem0)
            if j == _SPT - 1:
                @pl.when(p < _NSLAB // 2 - 1)
                def _():
                    nb = base + (2 * p + 2) * _SPT
                    pltpu.make_async_copy(srcp_hbm.at[pl.ds(nb, _SPT)],
                                          srcb0, isem0).wait()
                    pltpu.make_async_copy(dstp_hbm.at[pl.ds(nb, _SPT)],
                                          dstb0, isem0).wait()
                    chunk_work(srcb1, dstb1, _SPT - 1, (_SPT - 1) % 2,
                               (srcb0, 0))

                @pl.when(p >= _NSLAB // 2 - 1)
                def _():
                    chunk_work(srcb1, dstb1, _SPT - 1, (_SPT - 1) % 2, None)
            else:
                chunk_work(srcb1, dstb1, j, j % 2, (srcb1, j + 1))
        return 0

    lax.fori_loop(0, _NSLAB // 2, pair, 0)

    pltpu.make_async_copy(gbuf0, acc.at[dumpb], ssem0).wait()
    pltpu.make_async_copy(gbuf1, acc.at[dumpb], ssem1).wait()
    if with_den:
        pltpu.make_async_copy(wbuf0, dacc.at[dumpb], dsem0).wait()
        pltpu.make_async_copy(wbuf1, dacc.at[dumpb], dsem1).wait()
    plsc.subcore_barrier()
    pltpu.sync_copy(acc.at[pl.ds(sid * 640, 640)],
                    out_hbm.at[cid, pl.ds(sid * 640, 640)])
    if with_den:
        pltpu.sync_copy(dacc.at[pl.ds(sid * 640, 640)],
                        den_hbm.at[cid, pl.ds(sid * 640, 640)])


def _sc_edge_pass(h, a_src, a_dst, amax, srcp, dstp, with_den=False,
                  weighted=True):
    width = h.shape[1]
    out_type = jax.ShapeDtypeStruct((_NC, _ACC_ROWS, width), jnp.float32)
    if with_den:
        out_type = [out_type,
                    jax.ShapeDtypeStruct((_NC, _ACC_ROWS), jnp.float32)]
    scratch = [
        pltpu.VMEM((_SPT, _CHUNK), jnp.int32),
        pltpu.VMEM((_SPT, _CHUNK), jnp.int32),
        pltpu.VMEM((_SPT, _CHUNK), jnp.int32),
        pltpu.VMEM((_SPT, _CHUNK), jnp.int32),
        pltpu.VMEM((_CHUNK, width), jnp.float32),
        pltpu.VMEM((_CHUNK, width), jnp.float32),
        pltpu.VMEM((N_NODES,), jnp.float32),
        pltpu.VMEM((N_NODES,), jnp.float32),
        pltpu.VMEM((16,), jnp.float32),
        pltpu.VMEM((_CHUNK,), jnp.float32),
        pltpu.VMEM((_CHUNK,), jnp.float32),
        pltpu.VMEM((_CHUNK,), jnp.int32),
        pltpu.MemorySpace.VMEM_SHARED((_ACC_ROWS, width), jnp.float32),
        pltpu.SemaphoreType.DMA,
        pltpu.SemaphoreType.DMA,
        pltpu.SemaphoreType.DMA,
        pltpu.SemaphoreType.DMA,
        pltpu.SemaphoreType.DMA,
        pltpu.SemaphoreType.DMA,
    ]
    if with_den:
        scratch += [
            pltpu.VMEM((640,), jnp.float32),
            pltpu.MemorySpace.VMEM_SHARED((_ACC_ROWS,), jnp.float32),
            pltpu.SemaphoreType.DMA,
            pltpu.SemaphoreType.DMA,
        ]
    f = pl.kernel(
        functools.partial(_sc_edge_body, nf=width // 16, with_den=with_den,
                          weighted=weighted),
        out_type=out_type,
        mesh=_SC_MESH,
        scratch_types=scratch,
        compiler_params=pltpu.CompilerParams(needs_layout_passes=False),
    )
    return f(h, a_src, a_dst, amax, srcp, dstp)


def _pad_edges(src, dst):
    npad = _E_PAD - N_EDGES
    pad_dst = N_NODES + (jnp.arange(npad, dtype=jnp.int32)
                         % (_ACC_ROWS - N_NODES))
    srcp = jnp.concatenate(
        [src, jnp.zeros((npad,), jnp.int32)]).reshape(_N_CHUNKS, _CHUNK)
    dstp = jnp.concatenate([dst, pad_dst]).reshape(_N_CHUNKS, _CHUNK)
    return srcp, dstp


def _bn_eval(x, g, b):
    return x * (_BN_SCALE * g) + b


def _ln(x, g, b):
    m = jnp.mean(x, axis=-1, keepdims=True)
    v = jnp.mean((x - m) ** 2, axis=-1, keepdims=True)
    return (x - m) / jnp.sqrt(v + 1e-5) * g + b


def _mlp(x, p, pre):
    x = x @ p[pre + '_l0_w'].T + p[pre + '_l0_b']
    x = _bn_eval(x, p[pre + '_bn0_g'], p[pre + '_bn0_b'])
    x = jax.nn.relu(x)
    x = x @ p[pre + '_l1_w'].T + p[pre + '_l1_b']
    x = _bn_eval(x, p[pre + '_bn1_g'], p[pre + '_bn1_b'])
    return x


_Z_NODE = None


def _gin_conv(x, srcp, dstp, zn, z16, p, pre):
    parts = _sc_edge_pass(x, zn, zn, z16, srcp, dstp, weighted=False)
    return _mlp(x + parts[0, :N_NODES] + parts[1, :N_NODES], p, pre)


def _after(dep, *xs):
    return lax.optimization_barrier((dep, xs))[1]


def _gat_pre(x, w, att_src, att_dst):
    h = x @ w.T
    a_src = jnp.sum(h * att_src, axis=-1)
    a_dst = jnp.sum(h * att_dst, axis=-1)
    amax = jnp.max(a_src)
    return h, a_src, a_dst, amax


def _gat_post(num, den, h, a_src, a_dst, amax, bias):
    al = jax.nn.leaky_relu(a_src + a_dst, 0.2)
    c = jax.nn.leaky_relu(a_dst + amax, 0.2)
    wl = jnp.exp(al - c)
    ntot = num + wl[:, None] * h
    dtot = den + wl
    return ntot / (dtot + 1e-16)[:, None] + bias


def _mab(Q_in, K, V, p, pre):
    Q = Q_in @ p[pre + '_fcq_w'].T + p[pre + '_fcq_b']

    def split_heads(t):
        return jnp.concatenate(jnp.split(t, NUM_HEADS, axis=2), axis=0)

    Q_ = split_heads(Q)
    K_ = split_heads(K)
    V_ = split_heads(V)
    score = jnp.einsum('bqd,bkd->bqk', Q_, K_) / math.sqrt(HID)
    A = jax.nn.softmax(score, axis=-1)
    out = Q_ + jnp.einsum('bqk,bkd->bqd', A, V_)
    out = jnp.concatenate(jnp.split(out, NUM_HEADS, axis=0), axis=2)
    out = _ln(out, p[pre + '_ln0_g'], p[pre + '_ln0_b'])
    out = out + jax.nn.relu(out @ p[pre + '_fco_w'].T + p[pre + '_fco_b'])
    out = _ln(out, p[pre + '_ln1_g'], p[pre + '_ln1_b'])
    return out


def kernel(x, edge_index, batch, params):
    p = params
    src = edge_index[0]
    dst = edge_index[1]
    srcp, dstp = _pad_edges(src, dst)
    zn = jnp.zeros((N_NODES,), jnp.float32)
    z16 = jnp.zeros((16,), jnp.float32)
    h = _linear_tc(x, p['enc_w'], p['enc_b'])
    h = _gin_conv(h, srcp, dstp, zn, z16, p, 'c1')
    h = jax.nn.relu(h)
    h = _gin_conv(h, srcp, dstp, zn, z16, p, 'c2')
    xg = h @ p['gmt_lin1_w'].T + p['gmt_lin1_b']
    hK, aKs, aKd, amaxK = _gat_pre(xg, p['gatk_lin_w'], p['gatk_att_src'],
                                   p['gatk_att_dst'])
    hV, aVs, aVd, amaxV = _gat_pre(xg, p['gatv_lin_w'], p['gatv_att_src'],
                                   p['gatv_att_dst'])
    amaxK16 = jnp.broadcast_to(amaxK, (16,))
    amaxV16 = jnp.broadcast_to(amaxV, (16,))
    numK, wsK = _sc_edge_pass(hK, aKs, aKd, amaxK16, srcp, dstp,
                              with_den=True)
    hV_, aVs_, aVd_, amaxV16_ = _after((numK, wsK), hV, aVs, aVd, amaxV16)
    numV, wsV = _sc_edge_pass(hV_, aVs_, aVd_, amaxV16_, srcp, dstp,
                              with_den=True)
    nK = numK[0, :N_NODES] + numK[1, :N_NODES]
    nV = numV[0, :N_NODES] + numV[1, :N_NODES]
    denK = wsK[0, :N_NODES] + wsK[1, :N_NODES]
    denV = wsV[0, :N_NODES] + wsV[1, :N_NODES]
    K = _gat_post(nK, denK, hK, aKs, aKd, amaxK, p['gatk_bias'])[None]
    V = _gat_post(nV, denV, hV, aVs, aVd, amaxV, p['gatv_bias'])[None]
    S = jnp.broadcast_to(p['pma1_S'], (1, SEEDS1, HID))
    bx = _mab(S, K, V, p, 'mab1')
    K2 = bx @ p['mab2_lk_w'].T + p['mab2_lk_b']
    V2 = bx @ p['mab2_lv_w'].T + p['mab2_lv_b']
    bx = _mab(bx, K2, V2, p, 'mab2')
    K3 = bx @ p['mab3_lk_w'].T + p['mab3_lk_b']
    V3 = bx @ p['mab3_lv_w'].T + p['mab3_lv_b']
    bx = _mab(p['pma2_S'], K3, V3, p, 'mab3')
    out = bx[:, 0, :] @ p['gmt_lin2_w'].T + p['gmt_lin2_b']
    out = out @ p['clf_w'].T + p['clf_b']
    return out


# speedup vs baseline: 11.0999x; 1.0006x over previous
import functools
import math

import jax
import jax.numpy as jnp
from jax import lax
from jax.experimental import pallas as pl
from jax.experimental.pallas import tpu as pltpu
from jax.experimental.pallas import tpu_sc as plsc

N_NODES = 10000
N_EDGES = 320000
IN_CH = 128
HID = 128
OUT_CH = 64
NUM_HEADS = 4
SEEDS1 = 75

_BN_SCALE = 1.0 / math.sqrt(1.0 + 1e-5)


def _linear_body(x_ref, w_ref, b_ref, o_ref):
    o_ref[...] = lax.dot_general(
        x_ref[...], w_ref[...], (((1,), (1,)), ((), ())),
        preferred_element_type=jnp.float32) + b_ref[...]


def _linear_tc(x, w, b, blk=1000):
    n, _ = x.shape
    od, idim = w.shape
    return pl.pallas_call(
        _linear_body,
        grid=(n // blk,),
        in_specs=[
            pl.BlockSpec((blk, idim), lambda i: (i, 0)),
            pl.BlockSpec((od, idim), lambda i: (0, 0)),
            pl.BlockSpec((1, od), lambda i: (0, 0)),
        ],
        out_specs=pl.BlockSpec((blk, od), lambda i: (i, 0)),
        out_shape=jax.ShapeDtypeStruct((n, od), jnp.float32),
    )(x, w, b[None])


_NC = 2
_NS = 16
_NW = _NC * _NS
_CHUNK = 64
_SPT = 16
_NSLAB = 10
_CPT = _SPT * _NSLAB
_N_CHUNKS = _NW * _CPT
_E_PAD = _N_CHUNKS * _CHUNK
_ACC_ROWS = 10240

_SC_MESH = plsc.VectorSubcoreMesh(core_axis_name="c", subcore_axis_name="s")


def _edge_weights(sb, db, asb, adb, amax, j, t):
    s16 = sb.at[j][pl.ds(16 * t, 16)]
    d16 = db.at[j][pl.ds(16 * t, 16)]
    a_s = plsc.load_gather(asb, [s16])
    a_d = plsc.load_gather(adb, [d16])
    al = a_s + a_d
    al = jnp.maximum(al, 0.2 * al)
    c = a_d + amax
    c = jnp.maximum(c, 0.2 * c)
    return jnp.exp(al - c)


def _sc_edge_body(h_hbm, asrc_hbm, adst_hbm, amax_hbm, srcp_hbm, dstp_hbm,
                  out_hbm, *rest, nf, with_den, weighted):
    if with_den:
        den_hbm = rest[0]
        rest = rest[1:]
    (srcb0, dstb0, srcb1, dstb1, gbuf0, gbuf1, asb, adb, amaxb,
     wbuf0, wbuf1, dumpb, acc, gsem0, gsem1, ssem0, ssem1,
     isem0, isem1) = rest[:19]
    if with_den:
        dzb, dacc, dsem0, dsem1 = rest[19:23]
    cid = lax.axis_index("c")
    sid = lax.axis_index("s")
    wid = cid * _NS + sid
    base = wid * _CPT
    z16 = jnp.zeros((16,), jnp.float32)
    lane16 = lax.iota(jnp.int32, 16)

    def zrow(i, _):
        for t in range(nf):
            gbuf0.at[i][pl.ds(16 * t, 16)] = z16
            gbuf1.at[i][pl.ds(16 * t, 16)] = z16
        return 0

    lax.fori_loop(0, _CHUNK, zrow, 0)
    for t in range(4):
        dumpb[pl.ds(16 * t, 16)] = N_NODES + 16 * t + lane16

    if with_den:
        for t in range(4):
            wbuf0[pl.ds(16 * t, 16)] = z16
            wbuf1[pl.ds(16 * t, 16)] = z16
        for t in range(40):
            dzb[pl.ds(16 * t, 16)] = z16

    def zacc(k, _):
        pltpu.sync_copy(gbuf0, acc.at[pl.ds(sid * 640 + k * 64, 64)])
        return 0

    lax.fori_loop(0, 10, zacc, 0)
    if with_den:
        pltpu.sync_copy(dzb, dacc.at[pl.ds(sid * 640, 640)])

    if weighted:
        pltpu.sync_copy(asrc_hbm, asb)
        pltpu.sync_copy(adst_hbm, adb)
        pltpu.sync_copy(amax_hbm, amaxb)
    pltpu.sync_copy(srcp_hbm.at[pl.ds(base, _SPT)], srcb0)
    pltpu.sync_copy(dstp_hbm.at[pl.ds(base, _SPT)], dstb0)
    plsc.subcore_barrier()
    amax = amaxb[...] if weighted else None

    pltpu.async_copy(gbuf0, acc.at[dumpb], ssem0, add=True)
    pltpu.async_copy(gbuf1, acc.at[dumpb], ssem1, add=True)
    if with_den:
        pltpu.async_copy(wbuf0, dacc.at[dumpb], dsem0, add=True)
        pltpu.async_copy(wbuf1, dacc.at[dumpb], dsem1, add=True)
    pltpu.async_copy(h_hbm.at[srcb0.at[0]], gbuf0, gsem0)

    if with_den:
        bufs = ((gbuf0, wbuf0, gsem0, ssem0, dsem0),
                (gbuf1, wbuf1, gsem1, ssem1, dsem1))
    else:
        bufs = ((gbuf0, wbuf0, gsem0, ssem0, None),
                (gbuf1, wbuf1, gsem1, ssem1, None))

    def chunk_work(sb, db, jj, q, nxt):
        gq, wq, gsem, ssem, dsem = bufs[q]
        go, _, gso, _, _ = bufs[1 - q]
        if with_den:
            pltpu.make_async_copy(wq, dacc.at[db.at[jj]], dsem).wait()
        if weighted:
            for t in range(4):
                wq[pl.ds(16 * t, 16)] = _edge_weights(sb, db, asb, adb,
                                                      amax, jj, t)
        pltpu.make_async_copy(h_hbm.at[sb.at[jj]], gq, gsem).wait()
        if nxt is not None:
            nsb, njj = nxt
            pltpu.async_copy(h_hbm.at[nsb.at[njj]], go, gso)
        pltpu.make_async_copy(gq, acc.at[db.at[jj]], ssem).wait()

        def scale(e, _):
            ev = jnp.broadcast_to(e, (16,)).astype(jnp.int32)
            we = plsc.load_gather(wq, [ev])
            for t in range(nf):
                gq.at[e][pl.ds(16 * t, 16)] = (
                    gq.at[e][pl.ds(16 * t, 16)] * we)
            return 0

        if weighted:
            lax.fori_loop(0, _CHUNK, scale, 0)
        pltpu.async_copy(gq, acc.at[db.at[jj]], ssem, add=True)
        if with_den:
            pltpu.async_copy(wq, dacc.at[db.at[jj]], dsem, add=True)

    def pair(p, _):
        a0 = base + 2 * p * _SPT
        b0 = a0 + _SPT
        for j in range(_SPT):
            if j == 3:
                pltpu.async_copy(srcp_hbm.at[pl.ds(b0, _SPT)], srcb1, isem1)
                pltpu.async_copy(dstp_hbm.at[pl.ds(b0, _SPT)], dstb1, isem1)
            if j == _SPT - 1:
                pltpu.make_async_copy(srcp_hbm.at[pl.ds(b0, _SPT)], srcb1,
                                      isem1).wait()
                pltpu.make_async_copy(dstp_hbm.at[pl.ds(b0, _SPT)], dstb1,
                                      isem1).wait()
                nxt = (srcb1, 0)
            else:
                nxt = (srcb0, j + 1)
            chunk_work(srcb0, dstb0, j, j % 2, nxt)
        for j in range(_SPT):
            if j == 3:
                @pl.when(p < _NSLAB // 2 - 1)
                def _():
                    nb = base + (2 * p + 2) * _SPT
                    pltpu.async_copy(srcp_hbm.at[pl.ds(nb, _SPT)], srcb0,
                                     isem0)
                    pltpu.async_copy(dstp_hbm.at[pl.ds(nb, _SPT)], dstb0,
                                     isem0)
            if j == _SPT - 1:
                @pl.when(p < _NSLAB // 2 - 1)
                def _():
                    nb = base + (2 * p + 2) * _SPT
                    pltpu.make_async_copy(srcp_hbm.at[pl.ds(nb, _SPT)],
                                          srcb0, isem0).wait()
                    pltpu.make_async_copy(dstp_hbm.at[pl.ds(nb, _SPT)],
                                          dstb0, isem0).wait()
                    chunk_work(srcb1, dstb1, _SPT - 1, (_SPT - 1) % 2,
                               (srcb0, 0))

                @pl.when(p >= _NSLAB // 2 - 1)
                def _():
                    chunk_work(srcb1, dstb1, _SPT - 1, (_SPT - 1) % 2, None)
            else:
                chunk_work(srcb1, dstb1, j, j % 2, (srcb1, j + 1))
        return 0

    lax.fori_loop(0, _NSLAB // 2, pair, 0)

    pltpu.make_async_copy(gbuf0, acc.at[dumpb], ssem0).wait()
    pltpu.make_async_copy(gbuf1, acc.at[dumpb], ssem1).wait()
    if with_den:
        pltpu.make_async_copy(wbuf0, dacc.at[dumpb], dsem0).wait()
        pltpu.make_async_copy(wbuf1, dacc.at[dumpb], dsem1).wait()
    plsc.subcore_barrier()
    pltpu.sync_copy(acc.at[pl.ds(sid * 640, 640)],
                    out_hbm.at[cid, pl.ds(sid * 640, 640)])
    if with_den:
        pltpu.sync_copy(dacc.at[pl.ds(sid * 640, 640)],
                        den_hbm.at[cid, pl.ds(sid * 640, 640)])


def _sc_edge_pass(h, a_src, a_dst, amax, srcp, dstp, with_den=False,
                  weighted=True):
    width = h.shape[1]
    out_type = jax.ShapeDtypeStruct((_NC, _ACC_ROWS, width), jnp.float32)
    if with_den:
        out_type = [out_type,
                    jax.ShapeDtypeStruct((_NC, _ACC_ROWS), jnp.float32)]
    scratch = [
        pltpu.VMEM((_SPT, _CHUNK), jnp.int32),
        pltpu.VMEM((_SPT, _CHUNK), jnp.int32),
        pltpu.VMEM((_SPT, _CHUNK), jnp.int32),
        pltpu.VMEM((_SPT, _CHUNK), jnp.int32),
        pltpu.VMEM((_CHUNK, width), jnp.float32),
        pltpu.VMEM((_CHUNK, width), jnp.float32),
        pltpu.VMEM((N_NODES,), jnp.float32),
        pltpu.VMEM((N_NODES,), jnp.float32),
        pltpu.VMEM((16,), jnp.float32),
        pltpu.VMEM((_CHUNK,), jnp.float32),
        pltpu.VMEM((_CHUNK,), jnp.float32),
        pltpu.VMEM((_CHUNK,), jnp.int32),
        pltpu.MemorySpace.VMEM_SHARED((_ACC_ROWS, width), jnp.float32),
        pltpu.SemaphoreType.DMA,
        pltpu.SemaphoreType.DMA,
        pltpu.SemaphoreType.DMA,
        pltpu.SemaphoreType.DMA,
        pltpu.SemaphoreType.DMA,
        pltpu.SemaphoreType.DMA,
    ]
    if with_den:
        scratch += [
            pltpu.VMEM((640,), jnp.float32),
            pltpu.MemorySpace.VMEM_SHARED((_ACC_ROWS,), jnp.float32),
            pltpu.SemaphoreType.DMA,
            pltpu.SemaphoreType.DMA,
        ]
    f = pl.kernel(
        functools.partial(_sc_edge_body, nf=width // 16, with_den=with_den,
                          weighted=weighted),
        out_type=out_type,
        mesh=_SC_MESH,
        scratch_types=scratch,
        compiler_params=pltpu.CompilerParams(needs_layout_passes=False),
    )
    return f(h, a_src, a_dst, amax, srcp, dstp)


def _pad_edges(src, dst):
    npad = _E_PAD - N_EDGES
    pad_dst = N_NODES + (jnp.arange(npad, dtype=jnp.int32)
                         % (_ACC_ROWS - N_NODES))
    srcp = jnp.concatenate(
        [src, jnp.zeros((npad,), jnp.int32)]).reshape(_N_CHUNKS, _CHUNK)
    dstp = jnp.concatenate([dst, pad_dst]).reshape(_N_CHUNKS, _CHUNK)
    return srcp, dstp


def _bn_eval(x, g, b):
    return x * (_BN_SCALE * g) + b


def _ln(x, g, b):
    m = jnp.mean(x, axis=-1, keepdims=True)
    v = jnp.mean((x - m) ** 2, axis=-1, keepdims=True)
    return (x - m) / jnp.sqrt(v + 1e-5) * g + b


def _mlp(x, p, pre):
    x = x @ p[pre + '_l0_w'].T + p[pre + '_l0_b']
    x = _bn_eval(x, p[pre + '_bn0_g'], p[pre + '_bn0_b'])
    x = jax.nn.relu(x)
    x = x @ p[pre + '_l1_w'].T + p[pre + '_l1_b']
    x = _bn_eval(x, p[pre + '_bn1_g'], p[pre + '_bn1_b'])
    return x


_Z_NODE = None


def _gin_conv(x, srcp, dstp, zn, z16, p, pre):
    parts = _sc_edge_pass(x, zn, zn, z16, srcp, dstp, weighted=False)
    return _mlp(x + parts[0, :N_NODES] + parts[1, :N_NODES], p, pre)


def _after(dep, *xs):
    return lax.optimization_barrier((dep, xs))[1]


def _gat_pre(x, w, att_src, att_dst):
    h = x @ w.T
    a_src = jnp.sum(h * att_src, axis=-1)
    a_dst = jnp.sum(h * att_dst, axis=-1)
    amax = jnp.max(a_src)
    return h, a_src, a_dst, amax


def _gat_post(num, den, h, a_src, a_dst, amax, bias):
    al = jax.nn.leaky_relu(a_src + a_dst, 0.2)
    c = jax.nn.leaky_relu(a_dst + amax, 0.2)
    wl = jnp.exp(al - c)
    ntot = num + wl[:, None] * h
    dtot = den + wl
    return ntot / (dtot + 1e-16)[:, None] + bias


def _mab(Q_in, K, V, p, pre):
    Q = Q_in @ p[pre + '_fcq_w'].T + p[pre + '_fcq_b']

    def split_heads(t):
        return jnp.concatenate(jnp.split(t, NUM_HEADS, axis=2), axis=0)

    Q_ = split_heads(Q)
    K_ = split_heads(K)
    V_ = split_heads(V)
    score = jnp.einsum('bqd,bkd->bqk', Q_, K_) / math.sqrt(HID)
    A = jax.nn.softmax(score, axis=-1)
    out = Q_ + jnp.einsum('bqk,bkd->bqd', A, V_)
    out = jnp.concatenate(jnp.split(out, NUM_HEADS, axis=0), axis=2)
    out = _ln(out, p[pre + '_ln0_g'], p[pre + '_ln0_b'])
    out = out + jax.nn.relu(out @ p[pre + '_fco_w'].T + p[pre + '_fco_b'])
    out = _ln(out, p[pre + '_ln1_g'], p[pre + '_ln1_b'])
    return out


def kernel(x, edge_index, batch, params):
    p = params
    src = edge_index[0]
    dst = edge_index[1]
    srcp, dstp = _pad_edges(src, dst)
    zn = jnp.zeros((N_NODES,), jnp.float32)
    z16 = jnp.zeros((16,), jnp.float32)
    h = _linear_tc(x, p['enc_w'], p['enc_b'])
    h = _gin_conv(h, srcp, dstp, zn, z16, p, 'c1')
    h = jax.nn.relu(h)
    h = _gin_conv(h, srcp, dstp, zn, z16, p, 'c2')
    xg = h @ p['gmt_lin1_w'].T + p['gmt_lin1_b']
    hK, aKs, aKd, amaxK = _gat_pre(xg, p['gatk_lin_w'], p['gatk_att_src'],
                                   p['gatk_att_dst'])
    hV, aVs, aVd, amaxV = _gat_pre(xg, p['gatv_lin_w'], p['gatv_att_src'],
                                   p['gatv_att_dst'])
    amaxK16 = jnp.broadcast_to(amaxK, (16,))
    amaxV16 = jnp.broadcast_to(amaxV, (16,))
    numK, wsK = _sc_edge_pass(hK, aKs, aKd, amaxK16, srcp, dstp,
                              with_den=True)
    hV_, aVs_, aVd_, amaxV16_ = _after((numK, wsK), hV, aVs, aVd, amaxV16)
    numV, wsV = _sc_edge_pass(hV_, aVs_, aVd_, amaxV16_, srcp, dstp,
                              with_den=True)
    nK = numK[0, :N_NODES] + numK[1, :N_NODES]
    nV = numV[0, :N_NODES] + numV[1, :N_NODES]
    denK = wsK[0, :N_NODES] + wsK[1, :N_NODES]
    denV = wsV[0, :N_NODES] + wsV[1, :N_NODES]
    K = _gat_post(nK, denK, hK, aKs, aKd, amaxK, p['gatk_bias'])[None]
    V = _gat_post(nV, denV, hV, aVs, aVd, amaxV, p['gatv_bias'])[None]
    S = jnp.broadcast_to(p['pma1_S'], (1, SEEDS1, HID))
    bx = _mab(S, K, V, p, 'mab1')
    K2 = bx @ p['mab2_lk_w'].T + p['mab2_lk_b']
    V2 = bx @ p['mab2_lv_w'].T + p['mab2_lv_b']
    bx = _mab(bx, K2, V2, p, 'mab2')
    K3 = bx @ p['mab3_lk_w'].T + p['mab3_lk_b']
    V3 = bx @ p['mab3_lv_w'].T + p['mab3_lv_b']
    bx = _mab(p['pma2_S'], K3, V3, p, 'mab3')
    out = bx[:, 0, :] @ p['gmt_lin2_w'].T + p['gmt_lin2_b']
    out = out @ p['clf_w'].T + p['clf_b']
    return out
